# Initial kernel scaffold; baseline (speedup 1.0000x reference)
import jax
import jax.numpy as jnp
from jax.experimental import pallas as pl


def kernel(x, edge_index, W_enc0, b_enc0, W_enc1, b_enc1, W_gn0, b_gn0, W_gn1, b_gn1, Wmu0, bmu0, Wls0, bls0, Wmu1, bmu1, Wls1, bls1, eps0, eps1):
    raise NotImplementedError("write your pallas kernel here")


# speedup vs baseline: 2.1235x; 2.1235x over previous
import functools

import jax
import jax.numpy as jnp
from jax import lax
from jax.experimental import pallas as pl
from jax.experimental.pallas import tpu as pltpu
from jax.experimental.pallas import tpu_sc as plsc

N = 10000
E = 320000
F = 128
NC = 2
NS = 16
NW = NC * NS
NPAD = 10240
HALF = NPAD // 2
RPT = HALF // NS
ETI = E // NS
C = 80
NCH = ETI // C
EPT = E // NW
CM = 40
NCHM = EPT // CM
RB = 2000

_F32 = jnp.float32


def _mesh():
    return plsc.VectorSubcoreMesh(core_axis_name="c", subcore_axis_name="s")


def _mask_indices(c, sidx_v, didx_v, nch, cw):
    base = c * HALF

    def mrow(j, carry):
        for k in range(cw // 16):
            sl = pl.ds(k * 16, 16)
            t = didx_v[j, sl] - base
            valid = (t >= 0) & (t < HALF)
            didx_v[j, sl] = jnp.where(valid, t, -1)
            sidx_v[j, sl] = jnp.where(valid, sidx_v[j, sl], -1)
        return carry

    lax.fori_loop(0, nch, mrow, 0)


def _deg_body(sidx, didx, ones_hbm, zeros16, out,
              ones_v, oidx_v, iidx_v, degin_sh, degout_sh):
    c = lax.axis_index("c")
    s = lax.axis_index("s")
    base = c * HALF
    pltpu.sync_copy(ones_hbm, ones_v)
    pltpu.sync_copy(sidx.at[s], oidx_v)
    pltpu.sync_copy(didx.at[s], iidx_v)

    def mrow(j, carry):
        for k in range(C // 16):
            sl = pl.ds(k * 16, 16)
            ti = iidx_v[j, sl] - base
            iidx_v[j, sl] = jnp.where((ti >= 0) & (ti < HALF), ti, -1)
            to = oidx_v[j, sl] - base
            oidx_v[j, sl] = jnp.where((to >= 0) & (to < HALF), to, -1)
        return carry

    lax.fori_loop(0, NCH, mrow, 0)
    pltpu.sync_copy(zeros16, degin_sh.at[pl.ds(s * RPT, RPT)])
    pltpu.sync_copy(zeros16, degout_sh.at[pl.ds(s * RPT, RPT)])
    plsc.subcore_barrier()

    def body(j, carry):
        pltpu.sync_copy(
            ones_v, degin_sh.at[plsc.Indices(iidx_v.at[j], ignored_value=-1)],
            add=True)
        pltpu.sync_copy(
            ones_v, degout_sh.at[plsc.Indices(oidx_v.at[j], ignored_value=-1)],
            add=True)
        return carry

    lax.fori_loop(0, NCH, body, 0)
    plsc.subcore_barrier()
    pltpu.sync_copy(degin_sh.at[pl.ds(s * RPT, RPT)],
                    out.at[0, pl.ds(base + s * RPT, RPT)])
    pltpu.sync_copy(degout_sh.at[pl.ds(s * RPT, RPT)],
                    out.at[1, pl.ds(base + s * RPT, RPT)])


def _deg_call(sidxT, didxT):
    ones16 = jnp.ones((C, 16), _F32)
    zeros16 = jnp.zeros((RPT, 16), _F32)
    fn = pl.kernel(
        _deg_body,
        out_type=jax.ShapeDtypeStruct((2, NPAD, 16), _F32),
        mesh=_mesh(),
        scratch_types=[
            pltpu.VMEM((C, 16), _F32),
            pltpu.VMEM((NCH, C), jnp.int32),
            pltpu.VMEM((NCH, C), jnp.int32),
            pltpu.VMEM_SHARED((HALF, 16), _F32),
            pltpu.VMEM_SHARED((HALF, 16), _F32),
        ],
    )
    return fn(sidxT, didxT, ones16, zeros16)


def _conv_body(with_a, *refs):
    if with_a:
        (sidx, didx, tbl, a_hbm, zerosF, out,
         sidx_v, didx_v, rows_v, a_v, agg_sh, sem) = refs
    else:
        (sidx, didx, tbl, zerosF, out,
         sidx_v, didx_v, rows_v, agg_sh, sem) = refs
    c = lax.axis_index("c")
    s = lax.axis_index("s")
    pltpu.sync_copy(sidx.at[s], sidx_v)
    pltpu.sync_copy(didx.at[s], didx_v)
    _mask_indices(c, sidx_v, didx_v, NCH, C)
    pltpu.sync_copy(zerosF, agg_sh.at[pl.ds(s * RPT, RPT)])
    plsc.subcore_barrier()
    ebase = s * ETI

    def body(j, carry):
        pltpu.async_copy(
            tbl.at[plsc.Indices(sidx_v.at[j], ignored_value=-1)],
            rows_v, sem).wait()
        if with_a:
            pltpu.sync_copy(a_hbm.at[pl.ds(ebase + j * C, C)], a_v)

            def mul_body(e, cc):
                for f in range(F // 16):
                    sl = pl.ds(f * 16, 16)
                    rows_v[e, sl] = rows_v[e, sl] * a_v[e, sl]
                return cc

            lax.fori_loop(0, C, mul_body, 0)
        pltpu.sync_copy(
            rows_v, agg_sh.at[plsc.Indices(didx_v.at[j], ignored_value=-1)],
            add=True)
        return carry

    lax.fori_loop(0, NCH, body, 0)
    plsc.subcore_barrier()
    pltpu.sync_copy(agg_sh.at[pl.ds(s * RPT, RPT)],
                    out.at[pl.ds(c * HALF + s * RPT, RPT)])


def _conv_call(sidxT, didxT, tbl, a=None):
    zerosF = jnp.zeros((RPT, F), _F32)
    with_a = a is not None
    scratch = [
        pltpu.VMEM((NCH, C), jnp.int32),
        pltpu.VMEM((NCH, C), jnp.int32),
        pltpu.VMEM((C, F), _F32),
    ]
    if with_a:
        scratch.append(pltpu.VMEM((C, F), _F32))
    scratch += [
        pltpu.VMEM_SHARED((HALF, F), _F32),
        pltpu.SemaphoreType.DMA,
    ]
    fn = pl.kernel(
        functools.partial(_conv_body, with_a),
        out_type=jax.ShapeDtypeStruct((NPAD, F), _F32),
        mesh=_mesh(),
        scratch_types=scratch,
    )
    if with_a:
        return fn(sidxT, didxT, tbl, a, zerosF)
    return fn(sidxT, didxT, tbl, zerosF)


def _mlp_body(sidx, didx, ps_hbm, pd_hbm, eps0, eps1, a0_out, a1_out,
              sidx_v, didx_v, ps_v, pd_v, e0_v, e1_v, a0_v, a1_v, sem, sem2):
    c = lax.axis_index("c")
    s = lax.axis_index("s")
    w = s * NC + c
    pltpu.sync_copy(sidx.at[w], sidx_v)
    pltpu.sync_copy(didx.at[w], didx_v)
    ebase = w * EPT

    def body(j, carry):
        base = ebase + j * CM
        cp1 = pltpu.async_copy(ps_hbm.at[sidx_v.at[j]], ps_v, sem)
        cp2 = pltpu.async_copy(pd_hbm.at[didx_v.at[j]], pd_v, sem2)
        pltpu.sync_copy(eps0.at[pl.ds(base, CM)], e0_v)
        pltpu.sync_copy(eps1.at[pl.ds(base, CM)], e1_v)
        cp1.wait()
        cp2.wait()

        def e_body(e, cc):
            for f in range(F // 16):
                sl = pl.ds(f * 16, 16)
                s0 = ps_v[e, pl.ds(f * 16, 16)] + pd_v[e, pl.ds(f * 16, 16)]
                s1 = ps_v[e, pl.ds(128 + f * 16, 16)] + pd_v[e, pl.ds(128 + f * 16, 16)]
                s2 = ps_v[e, pl.ds(256 + f * 16, 16)] + pd_v[e, pl.ds(256 + f * 16, 16)]
                s3 = ps_v[e, pl.ds(384 + f * 16, 16)] + pd_v[e, pl.ds(384 + f * 16, 16)]
                a0_v[e, sl] = jnp.exp(s0) + jnp.exp(s1) * e0_v[e, sl]
                a1_v[e, sl] = jnp.exp(s2) + jnp.exp(s3) * e1_v[e, sl]
            return cc

        lax.fori_loop(0, CM, e_body, 0)
        pltpu.sync_copy(a0_v, a0_out.at[pl.ds(base, CM)])
        pltpu.sync_copy(a1_v, a1_out.at[pl.ds(base, CM)])
        return carry

    lax.fori_loop(0, NCHM, body, 0)


def _mlp_call(sidxW, didxW, ps, pd, eps0, eps1):
    fn = pl.kernel(
        _mlp_body,
        out_type=(jax.ShapeDtypeStruct((E, F), _F32),
                  jax.ShapeDtypeStruct((E, F), _F32)),
        mesh=_mesh(),
        scratch_types=[
            pltpu.VMEM((NCHM, CM), jnp.int32),
            pltpu.VMEM((NCHM, CM), jnp.int32),
            pltpu.VMEM((CM, 4 * F), _F32),
            pltpu.VMEM((CM, 4 * F), _F32),
            pltpu.VMEM((CM, F), _F32),
            pltpu.VMEM((CM, F), _F32),
            pltpu.VMEM((CM, F), _F32),
            pltpu.VMEM((CM, F), _F32),
            pltpu.SemaphoreType.DMA,
            pltpu.SemaphoreType.DMA,
        ],
    )
    return fn(sidxW, didxW, ps, pd, eps0, eps1)


def _ni_of(degp_blk):
    return lax.rsqrt(jnp.maximum(degp_blk[0][:, :1], 1.0))


def _no_of(degp_blk):
    return lax.rsqrt(jnp.maximum(degp_blk[1][:, :1], 1.0))


_DEG_SPEC = pl.BlockSpec((2, RB, 16), lambda i: (0, i, 0))
_AGG_SPEC = pl.BlockSpec((RB, F), lambda i: (i, 0))
_ROW_SPEC = pl.BlockSpec((RB, F), lambda i: (i, 0))


def _xs_body(x_ref, degp_ref, o_ref):
    o_ref[...] = x_ref[...] * _no_of(degp_ref)


def _xs_call(x, degp):
    return pl.pallas_call(
        _xs_body,
        grid=(N // RB,),
        in_specs=[_ROW_SPEC, _DEG_SPEC],
        out_specs=_ROW_SPEC,
        out_shape=jax.ShapeDtypeStruct((N, F), _F32),
    )(x, degp)


def _node_body(agg_ref, degp_ref, w_ref, b_ref, o_ref):
    t = agg_ref[...] * _ni_of(degp_ref)
    y = jnp.dot(t, w_ref[...], preferred_element_type=_F32) + b_ref[...]
    y = jnp.maximum(y, 0.0) * _no_of(degp_ref)
    o_ref[...] = y


def _node_call(agg, degp, w, b):
    return pl.pallas_call(
        _node_body,
        grid=(N // RB,),
        in_specs=[
            _AGG_SPEC,
            _DEG_SPEC,
            pl.BlockSpec((F, F), lambda i: (0, 0)),
            pl.BlockSpec((1, F), lambda i: (0, 0)),
        ],
        out_specs=_ROW_SPEC,
        out_shape=jax.ShapeDtypeStruct((N, F), _F32),
    )(agg, degp, w, b.reshape(1, F))


def _proj_body(agg_ref, degp_ref, w1_ref, b1_ref, ws_ref, wd_ref, bc_ref,
               ps_ref, pd_ref):
    t = agg_ref[...] * _ni_of(degp_ref)
    z = jnp.dot(t, w1_ref[...], preferred_element_type=_F32) + b1_ref[...]
    z = jnp.maximum(z, 0.0)
    ps_ref[...] = jnp.dot(z, ws_ref[...], preferred_element_type=_F32)
    pd_ref[...] = jnp.dot(z, wd_ref[...], preferred_element_type=_F32) + bc_ref[...]


def _proj_call(agg, degp, w1, b1, ws, wd, bc):
    spec512 = pl.BlockSpec((RB, 4 * F), lambda i: (i, 0))
    return pl.pallas_call(
        _proj_body,
        grid=(N // RB,),
        in_specs=[
            _AGG_SPEC,
            _DEG_SPEC,
            pl.BlockSpec((F, F), lambda i: (0, 0)),
            pl.BlockSpec((1, F), lambda i: (0, 0)),
            pl.BlockSpec((F, 4 * F), lambda i: (0, 0)),
            pl.BlockSpec((F, 4 * F), lambda i: (0, 0)),
            pl.BlockSpec((1, 4 * F), lambda i: (0, 0)),
        ],
        out_specs=(spec512, spec512),
        out_shape=(jax.ShapeDtypeStruct((N, 4 * F), _F32),
                   jax.ShapeDtypeStruct((N, 4 * F), _F32)),
    )(agg, degp, w1, b1.reshape(1, F), ws, wd, bc.reshape(1, 4 * F))


def _final_body(agg_ref, degp_ref, w_ref, b_ref, o_ref):
    t = agg_ref[...] * _ni_of(degp_ref)
    y = jnp.dot(t, w_ref[...], preferred_element_type=_F32) + b_ref[...]
    m = jnp.max(y, axis=-1, keepdims=True)
    ey = jnp.exp(y - m)
    o_ref[...] = ey / jnp.sum(ey, axis=-1, keepdims=True)


def _final_call(agg, degp, w, b):
    return pl.pallas_call(
        _final_body,
        grid=(N // RB,),
        in_specs=[
            _AGG_SPEC,
            _DEG_SPEC,
            pl.BlockSpec((F, F), lambda i: (0, 0)),
            pl.BlockSpec((1, F), lambda i: (0, 0)),
        ],
        out_specs=_ROW_SPEC,
        out_shape=jax.ShapeDtypeStruct((N, F), _F32),
    )(agg, degp, w, b.reshape(1, F))


def kernel(x, edge_index, W_enc0, b_enc0, W_enc1, b_enc1,
           W_gn0, b_gn0, W_gn1, b_gn1,
           Wmu0, bmu0, Wls0, bls0, Wmu1, bmu1, Wls1, bls1,
           eps0, eps1):
    src = edge_index[0]
    dst = edge_index[1]
    sidxT = src.reshape(NS, NCH, C)
    didxT = dst.reshape(NS, NCH, C)
    sidxW = src.reshape(NW, NCHM, CM)
    didxW = dst.reshape(NW, NCHM, CM)

    ws_cat = jnp.concatenate(
        [Wmu0[:F], Wls0[:F], Wmu1[:F], Wls1[:F]], axis=1)
    wd_cat = jnp.concatenate(
        [Wmu0[F:], Wls0[F:], Wmu1[F:], Wls1[F:]], axis=1)
    bc = jnp.concatenate([bmu0, bls0, bmu1, bls1])

    degp = _deg_call(sidxT, didxT)
    xs = _xs_call(x, degp)
    agg1 = _conv_call(sidxT, didxT, xs)
    z1s = _node_call(agg1, degp, W_enc0, b_enc0)
    agg2 = _conv_call(sidxT, didxT, z1s)
    ps, pd = _proj_call(agg2, degp, W_enc1, b_enc1, ws_cat, wd_cat, bc)
    a0, a1 = _mlp_call(sidxW, didxW, ps, pd, eps0, eps1)
    agg3 = _conv_call(sidxT, didxT, xs, a=a0)
    h1s = _node_call(agg3, degp, W_gn0, b_gn0)
    agg4 = _conv_call(sidxT, didxT, h1s, a=a1)
    return _final_call(agg4, degp, W_gn1, b_gn1)


# speedup vs baseline: 2.5524x; 1.2020x over previous
import functools

import jax
import jax.numpy as jnp
from jax import lax
from jax.experimental import pallas as pl
from jax.experimental.pallas import tpu as pltpu
from jax.experimental.pallas import tpu_sc as plsc

N = 10000
E = 320000
F = 128
NC = 2
NS = 16
NW = NC * NS
NPAD = 10240
HALF = NPAD // 2
RPT = HALF // NS
ETI = E // NS
C = 80
NCH = ETI // C
EPT = E // NW
CM = 40
NCHM = EPT // CM
RB = 2000

_F32 = jnp.float32


def _mesh():
    return plsc.VectorSubcoreMesh(core_axis_name="c", subcore_axis_name="s")


def _mask_indices(c, sidx_v, didx_v, nch, cw):
    base = c * HALF

    def mrow(j, carry):
        for k in range(cw // 16):
            sl = pl.ds(k * 16, 16)
            t = didx_v[j, sl] - base
            valid = (t >= 0) & (t < HALF)
            didx_v[j, sl] = jnp.where(valid, t, -1)
            sidx_v[j, sl] = jnp.where(valid, sidx_v[j, sl], -1)
        return carry

    lax.fori_loop(0, nch, mrow, 0)


def _deg_body(sidx, didx, ones_hbm, zeros16, out,
              ones_v, oidx_v, iidx_v, degin_sh, degout_sh):
    c = lax.axis_index("c")
    s = lax.axis_index("s")
    base = c * HALF
    pltpu.sync_copy(ones_hbm, ones_v)
    pltpu.sync_copy(sidx.at[s], oidx_v)
    pltpu.sync_copy(didx.at[s], iidx_v)

    def mrow(j, carry):
        for k in range(C // 16):
            sl = pl.ds(k * 16, 16)
            ti = iidx_v[j, sl] - base
            iidx_v[j, sl] = jnp.where((ti >= 0) & (ti < HALF), ti, -1)
            to = oidx_v[j, sl] - base
            oidx_v[j, sl] = jnp.where((to >= 0) & (to < HALF), to, -1)
        return carry

    lax.fori_loop(0, NCH, mrow, 0)
    pltpu.sync_copy(zeros16, degin_sh.at[pl.ds(s * RPT, RPT)])
    pltpu.sync_copy(zeros16, degout_sh.at[pl.ds(s * RPT, RPT)])
    plsc.subcore_barrier()

    def body(j, carry):
        pltpu.sync_copy(
            ones_v, degin_sh.at[plsc.Indices(iidx_v.at[j], ignored_value=-1)],
            add=True)
        pltpu.sync_copy(
            ones_v, degout_sh.at[plsc.Indices(oidx_v.at[j], ignored_value=-1)],
            add=True)
        return carry

    lax.fori_loop(0, NCH, body, 0)
    plsc.subcore_barrier()
    pltpu.sync_copy(degin_sh.at[pl.ds(s * RPT, RPT)],
                    out.at[0, pl.ds(base + s * RPT, RPT)])
    pltpu.sync_copy(degout_sh.at[pl.ds(s * RPT, RPT)],
                    out.at[1, pl.ds(base + s * RPT, RPT)])


def _deg_call(sidxT, didxT):
    ones16 = jnp.ones((C, 16), _F32)
    zeros16 = jnp.zeros((RPT, 16), _F32)
    fn = pl.kernel(
        _deg_body,
        out_type=jax.ShapeDtypeStruct((2, NPAD, 16), _F32),
        mesh=_mesh(),
        scratch_types=[
            pltpu.VMEM((C, 16), _F32),
            pltpu.VMEM((NCH, C), jnp.int32),
            pltpu.VMEM((NCH, C), jnp.int32),
            pltpu.VMEM_SHARED((HALF, 16), _F32),
            pltpu.VMEM_SHARED((HALF, 16), _F32),
        ],
    )
    return fn(sidxT, didxT, ones16, zeros16)


def _conv_body(with_a, *refs):
    if with_a:
        (sidx, didx, tbl, a_hbm, zerosF, out,
         sidx_v, didx_v, rows_v, a_v, agg_sh, sem) = refs
    else:
        (sidx, didx, tbl, zerosF, out,
         sidx_v, didx_v, rows_v, agg_sh, sem) = refs
    c = lax.axis_index("c")
    s = lax.axis_index("s")
    pltpu.sync_copy(sidx.at[s], sidx_v)
    pltpu.sync_copy(didx.at[s], didx_v)
    _mask_indices(c, sidx_v, didx_v, NCH, C)
    pltpu.sync_copy(zerosF, agg_sh.at[pl.ds(s * RPT, RPT)])
    plsc.subcore_barrier()
    ebase = s * ETI

    def body(j, carry):
        pltpu.async_copy(
            tbl.at[plsc.Indices(sidx_v.at[j], ignored_value=-1)],
            rows_v, sem).wait()
        if with_a:
            pltpu.sync_copy(a_hbm.at[pl.ds(ebase + j * C, C)], a_v)

            def mul_body(e, cc):
                for f in range(F // 16):
                    sl = pl.ds(f * 16, 16)
                    rows_v[e, sl] = rows_v[e, sl] * a_v[e, sl]
                return cc

            lax.fori_loop(0, C, mul_body, 0)
        pltpu.sync_copy(
            rows_v, agg_sh.at[plsc.Indices(didx_v.at[j], ignored_value=-1)],
            add=True)
        return carry

    lax.fori_loop(0, NCH, body, 0)
    plsc.subcore_barrier()
    pltpu.sync_copy(agg_sh.at[pl.ds(s * RPT, RPT)],
                    out.at[pl.ds(c * HALF + s * RPT, RPT)])


def _conv_call(sidxT, didxT, tbl, a=None):
    zerosF = jnp.zeros((RPT, F), _F32)
    with_a = a is not None
    scratch = [
        pltpu.VMEM((NCH, C), jnp.int32),
        pltpu.VMEM((NCH, C), jnp.int32),
        pltpu.VMEM((C, F), _F32),
    ]
    if with_a:
        scratch.append(pltpu.VMEM((C, F), _F32))
    scratch += [
        pltpu.VMEM_SHARED((HALF, F), _F32),
        pltpu.SemaphoreType.DMA,
    ]
    fn = pl.kernel(
        functools.partial(_conv_body, with_a),
        out_type=jax.ShapeDtypeStruct((NPAD, F), _F32),
        mesh=_mesh(),
        scratch_types=scratch,
    )
    if with_a:
        return fn(sidxT, didxT, tbl, a, zerosF)
    return fn(sidxT, didxT, tbl, zerosF)


def _mlp_chunk(eps, a_out, idx_v, rows_v, e_v, a_v, ebase, g):
    base = ebase + g * CM

    def e_body(e, cc):
        for f in range(F // 16):
            sl = pl.ds(f * 16, 16)
            em = rows_v[e, pl.ds(f * 16, 16)] * rows_v[CM + e, pl.ds(f * 16, 16)]
            es = rows_v[e, pl.ds(128 + f * 16, 16)] * rows_v[CM + e, pl.ds(128 + f * 16, 16)]
            a_v[e, sl] = em + es * e_v[e, sl]
        return cc

    lax.fori_loop(0, CM, e_body, 0)
    pltpu.sync_copy(a_v, a_out.at[pl.ds(base, CM)])


def _mlp_body(sdidx, p01_hbm, p23_hbm, eps0, eps1, a0_out, a1_out,
              idx_v, rows0_v, rows1_v, e0_v, e1_v, a_v, sem0, sem1, sem2, sem3):
    c = lax.axis_index("c")
    s = lax.axis_index("s")
    w = s * NC + c
    pltpu.sync_copy(sdidx.at[w], idx_v)
    ebase = w * EPT

    for p_hbm, eps, a_out in ((p01_hbm, eps0, a0_out), (p23_hbm, eps1, a1_out)):
        def issue(g, rows_v, sem, e_v, esem):
            pltpu.async_copy(p_hbm.at[plsc.Indices(idx_v.at[g])], rows_v, sem)
            pltpu.async_copy(eps.at[pl.ds(ebase + g * CM, CM)], e_v, esem)

        def wait(g, rows_v, sem, e_v, esem):
            pltpu.make_async_copy(
                p_hbm.at[plsc.Indices(idx_v.at[g])], rows_v, sem).wait()
            pltpu.make_async_copy(
                eps.at[pl.ds(ebase + g * CM, CM)], e_v, esem).wait()

        issue(0, rows0_v, sem0, e0_v, sem2)

        def body(g2, carry):
            g = 2 * g2
            issue(g + 1, rows1_v, sem1, e1_v, sem3)
            wait(g, rows0_v, sem0, e0_v, sem2)
            _mlp_chunk(eps, a_out, idx_v, rows0_v, e0_v, a_v, ebase, g)

            @pl.when(g + 2 < NCHM)
            def _():
                issue(g + 2, rows0_v, sem0, e0_v, sem2)

            wait(g + 1, rows1_v, sem1, e1_v, sem3)
            _mlp_chunk(eps, a_out, idx_v, rows1_v, e1_v, a_v, ebase, g + 1)
            return carry

        lax.fori_loop(0, NCHM // 2, body, 0)


def _mlp_call(sdidx, p01, p23, eps0, eps1):
    fn = pl.kernel(
        _mlp_body,
        out_type=(jax.ShapeDtypeStruct((E, F), _F32),
                  jax.ShapeDtypeStruct((E, F), _F32)),
        mesh=_mesh(),
        scratch_types=[
            pltpu.VMEM((NCHM, 2 * CM), jnp.int32),
            pltpu.VMEM((2 * CM, 2 * F), _F32),
            pltpu.VMEM((2 * CM, 2 * F), _F32),
            pltpu.VMEM((CM, F), _F32),
            pltpu.VMEM((CM, F), _F32),
            pltpu.VMEM((CM, F), _F32),
            pltpu.SemaphoreType.DMA,
            pltpu.SemaphoreType.DMA,
            pltpu.SemaphoreType.DMA,
            pltpu.SemaphoreType.DMA,
        ],
    )
    return fn(sdidx, p01, p23, eps0, eps1)


def _ni_of(degp_blk):
    return lax.rsqrt(jnp.maximum(degp_blk[0][:, :1], 1.0))


def _no_of(degp_blk):
    return lax.rsqrt(jnp.maximum(degp_blk[1][:, :1], 1.0))


_DEG_SPEC = pl.BlockSpec((2, RB, 16), lambda i: (0, i, 0))
_AGG_SPEC = pl.BlockSpec((RB, F), lambda i: (i, 0))
_ROW_SPEC = pl.BlockSpec((RB, F), lambda i: (i, 0))


def _xs_body(x_ref, degp_ref, o_ref):
    o_ref[...] = x_ref[...] * _no_of(degp_ref)


def _xs_call(x, degp):
    return pl.pallas_call(
        _xs_body,
        grid=(N // RB,),
        in_specs=[_ROW_SPEC, _DEG_SPEC],
        out_specs=_ROW_SPEC,
        out_shape=jax.ShapeDtypeStruct((N, F), _F32),
    )(x, degp)


def _node_body(agg_ref, degp_ref, w_ref, b_ref, o_ref):
    t = agg_ref[...] * _ni_of(degp_ref)
    y = jnp.dot(t, w_ref[...], preferred_element_type=_F32) + b_ref[...]
    y = jnp.maximum(y, 0.0) * _no_of(degp_ref)
    o_ref[...] = y


def _node_call(agg, degp, w, b):
    return pl.pallas_call(
        _node_body,
        grid=(N // RB,),
        in_specs=[
            _AGG_SPEC,
            _DEG_SPEC,
            pl.BlockSpec((F, F), lambda i: (0, 0)),
            pl.BlockSpec((1, F), lambda i: (0, 0)),
        ],
        out_specs=_ROW_SPEC,
        out_shape=jax.ShapeDtypeStruct((N, F), _F32),
    )(agg, degp, w, b.reshape(1, F))


def _proj_body(agg_ref, degp_ref, w1_ref, b1_ref, ws_ref, wd_ref, bc_ref,
               p01_ref, p23_ref):
    t = agg_ref[...] * _ni_of(degp_ref)
    z = jnp.dot(t, w1_ref[...], preferred_element_type=_F32) + b1_ref[...]
    z = jnp.maximum(z, 0.0)
    ps = jnp.exp(jnp.dot(z, ws_ref[...], preferred_element_type=_F32))
    pd = jnp.exp(
        jnp.dot(z, wd_ref[...], preferred_element_type=_F32) + bc_ref[...])
    p01_ref[0] = ps[:, :2 * F]
    p01_ref[1] = pd[:, :2 * F]
    p23_ref[0] = ps[:, 2 * F:]
    p23_ref[1] = pd[:, 2 * F:]


def _proj_call(agg, degp, w1, b1, ws, wd, bc):
    spec_p = pl.BlockSpec((2, RB, 2 * F), lambda i: (0, i, 0))
    return pl.pallas_call(
        _proj_body,
        grid=(N // RB,),
        in_specs=[
            _AGG_SPEC,
            _DEG_SPEC,
            pl.BlockSpec((F, F), lambda i: (0, 0)),
            pl.BlockSpec((1, F), lambda i: (0, 0)),
            pl.BlockSpec((F, 4 * F), lambda i: (0, 0)),
            pl.BlockSpec((F, 4 * F), lambda i: (0, 0)),
            pl.BlockSpec((1, 4 * F), lambda i: (0, 0)),
        ],
        out_specs=(spec_p, spec_p),
        out_shape=(jax.ShapeDtypeStruct((2, N, 2 * F), _F32),
                   jax.ShapeDtypeStruct((2, N, 2 * F), _F32)),
    )(agg, degp, w1, b1.reshape(1, F), ws, wd, bc.reshape(1, 4 * F))


def _final_body(agg_ref, degp_ref, w_ref, b_ref, o_ref):
    t = agg_ref[...] * _ni_of(degp_ref)
    y = jnp.dot(t, w_ref[...], preferred_element_type=_F32) + b_ref[...]
    m = jnp.max(y, axis=-1, keepdims=True)
    ey = jnp.exp(y - m)
    o_ref[...] = ey / jnp.sum(ey, axis=-1, keepdims=True)


def _final_call(agg, degp, w, b):
    return pl.pallas_call(
        _final_body,
        grid=(N // RB,),
        in_specs=[
            _AGG_SPEC,
            _DEG_SPEC,
            pl.BlockSpec((F, F), lambda i: (0, 0)),
            pl.BlockSpec((1, F), lambda i: (0, 0)),
        ],
        out_specs=_ROW_SPEC,
        out_shape=jax.ShapeDtypeStruct((N, F), _F32),
    )(agg, degp, w, b.reshape(1, F))


def kernel(x, edge_index, W_enc0, b_enc0, W_enc1, b_enc1,
           W_gn0, b_gn0, W_gn1, b_gn1,
           Wmu0, bmu0, Wls0, bls0, Wmu1, bmu1, Wls1, bls1,
           eps0, eps1):
    src = edge_index[0]
    dst = edge_index[1]
    sidxT = src.reshape(NS, NCH, C)
    didxT = dst.reshape(NS, NCH, C)
    sidxW = src.reshape(NW, NCHM, CM)
    didxW = dst.reshape(NW, NCHM, CM)
    sdidx = jnp.concatenate([sidxW, didxW + N], axis=-1)

    ws_cat = jnp.concatenate(
        [Wmu0[:F], Wls0[:F], Wmu1[:F], Wls1[:F]], axis=1)
    wd_cat = jnp.concatenate(
        [Wmu0[F:], Wls0[F:], Wmu1[F:], Wls1[F:]], axis=1)
    bc = jnp.concatenate([bmu0, bls0, bmu1, bls1])

    degp = _deg_call(sidxT, didxT)
    xs = _xs_call(x, degp)
    agg1 = _conv_call(sidxT, didxT, xs)
    z1s = _node_call(agg1, degp, W_enc0, b_enc0)
    agg2 = _conv_call(sidxT, didxT, z1s)
    p01, p23 = _proj_call(agg2, degp, W_enc1, b_enc1, ws_cat, wd_cat, bc)
    a0, a1 = _mlp_call(sdidx, p01.reshape(2 * N, 2 * F),
                       p23.reshape(2 * N, 2 * F), eps0, eps1)
    agg3 = _conv_call(sidxT, didxT, xs, a=a0)
    h1s = _node_call(agg3, degp, W_gn0, b_gn0)
    agg4 = _conv_call(sidxT, didxT, h1s, a=a1)
    return _final_call(agg4, degp, W_gn1, b_gn1)


# speedup vs baseline: 2.7114x; 1.0623x over previous
import functools

import jax
import jax.numpy as jnp
from jax import lax
from jax.experimental import pallas as pl
from jax.experimental.pallas import tpu as pltpu
from jax.experimental.pallas import tpu_sc as plsc

N = 10000
E = 320000
F = 128
NC = 2
NS = 16
NW = NC * NS
NPAD = 10240
HALF = NPAD // 2
RPT = HALF // NS
ETI = E // NS
C = 80
NCH = ETI // C
EPT = E // NW
CM = 40
NCHM = EPT // CM
RB = 2000

_F32 = jnp.float32


def _mesh():
    return plsc.VectorSubcoreMesh(core_axis_name="c", subcore_axis_name="s")


def _deg_body(sidx, didx, ones_hbm, zeros16, out,
              ones_v, oidx_v, iidx_v, degin_sh, degout_sh):
    c = lax.axis_index("c")
    s = lax.axis_index("s")
    base = c * HALF
    pltpu.sync_copy(ones_hbm, ones_v)
    pltpu.sync_copy(sidx.at[s], oidx_v)
    pltpu.sync_copy(didx.at[s], iidx_v)

    def mrow(j, carry):
        for k in range(C // 16):
            sl = pl.ds(k * 16, 16)
            ti = iidx_v[j, sl] - base
            iidx_v[j, sl] = jnp.where((ti >= 0) & (ti < HALF), ti, -1)
            to = oidx_v[j, sl] - base
            oidx_v[j, sl] = jnp.where((to >= 0) & (to < HALF), to, -1)
        return carry

    lax.fori_loop(0, NCH, mrow, 0)
    pltpu.sync_copy(zeros16, degin_sh.at[pl.ds(s * RPT, RPT)])
    pltpu.sync_copy(zeros16, degout_sh.at[pl.ds(s * RPT, RPT)])
    plsc.subcore_barrier()

    def body(j, carry):
        pltpu.sync_copy(
            ones_v, degin_sh.at[plsc.Indices(iidx_v.at[j], ignored_value=-1)],
            add=True)
        pltpu.sync_copy(
            ones_v, degout_sh.at[plsc.Indices(oidx_v.at[j], ignored_value=-1)],
            add=True)
        return carry

    lax.fori_loop(0, NCH, body, 0)
    plsc.subcore_barrier()
    pltpu.sync_copy(degin_sh.at[pl.ds(s * RPT, RPT)],
                    out.at[0, pl.ds(base + s * RPT, RPT)])
    pltpu.sync_copy(degout_sh.at[pl.ds(s * RPT, RPT)],
                    out.at[1, pl.ds(base + s * RPT, RPT)])


def _deg_call(sidxT, didxT):
    ones16 = jnp.ones((C, 16), _F32)
    zeros16 = jnp.zeros((RPT, 16), _F32)
    fn = pl.kernel(
        _deg_body,
        out_type=jax.ShapeDtypeStruct((2, NPAD, 16), _F32),
        mesh=_mesh(),
        scratch_types=[
            pltpu.VMEM((C, 16), _F32),
            pltpu.VMEM((NCH, C), jnp.int32),
            pltpu.VMEM((NCH, C), jnp.int32),
            pltpu.VMEM_SHARED((HALF, 16), _F32),
            pltpu.VMEM_SHARED((HALF, 16), _F32),
        ],
    )
    return fn(sidxT, didxT, ones16, zeros16)


def _conv_body(with_a, *refs):
    if with_a:
        (sidx, didx, tbl, a_hbm, zerosF, out,
         sidx_v, didx_v, rows0_v, rows1_v, a_v, agg_sh,
         semA, semB, semC) = refs
    else:
        (sidx, didx, tbl, zerosF, out,
         sidx_v, didx_v, rows0_v, rows1_v, agg_sh, semA, semB) = refs
    c = lax.axis_index("c")
    s = lax.axis_index("s")
    pltpu.sync_copy(sidx.at[s], sidx_v)
    pltpu.sync_copy(didx.at[s], didx_v)
    ebase = s * ETI
    base_n = c * HALF

    def mrow(j, carry):
        for k in range(C // 16):
            sl = pl.ds(k * 16, 16)
            t = didx_v[j, sl] - base_n
            valid = (t >= 0) & (t < HALF)
            didx_v[j, sl] = jnp.where(valid, t, -1)
            sidx_v[j, sl] = jnp.where(valid, sidx_v[j, sl], -1)
        return carry

    lax.fori_loop(0, NCH, mrow, 0)
    pltpu.sync_copy(zerosF, agg_sh.at[pl.ds(s * RPT, RPT)])
    plsc.subcore_barrier()

    def issue_rows(g, rows_v, sem):
        return pltpu.async_copy(
            tbl.at[plsc.Indices(sidx_v.at[g], ignored_value=-1)],
            rows_v, sem)

    def issue_a(g):
        return pltpu.async_copy(
            a_hbm.at[pl.ds(ebase + g * C, C)], a_v, semC)

    def mul(rows_v):
        def mul_body(e, cc):
            for f in range(F // 16):
                sl = pl.ds(f * 16, 16)
                a_v[e, sl] = rows_v[e, sl] * a_v[e, sl]
            return cc

        lax.fori_loop(0, C, mul_body, 0)

    def scatter(g, src_v):
        pltpu.sync_copy(
            src_v, agg_sh.at[plsc.Indices(didx_v.at[g], ignored_value=-1)],
            add=True)

    if with_a:
        def body_a(g, carry):
            issue_rows(g, rows0_v, semA).wait()
            issue_a(g).wait()
            mul(rows0_v)
            scatter(g, a_v)
            return carry

        lax.fori_loop(0, NCH, body_a, 0)
    else:
        def body(g2, carry):
            g = 2 * g2
            cp0 = issue_rows(g, rows0_v, semA)
            cp1 = issue_rows(g + 1, rows1_v, semB)
            cp0.wait()
            scatter(g, rows0_v)
            cp1.wait()
            scatter(g + 1, rows1_v)
            return carry

        lax.fori_loop(0, NCH // 2, body, 0)
    plsc.subcore_barrier()
    pltpu.sync_copy(agg_sh.at[pl.ds(s * RPT, RPT)],
                    out.at[pl.ds(c * HALF + s * RPT, RPT)])


def _conv_call(sidxT, didxT, tbl, a=None):
    zerosF = jnp.zeros((RPT, F), _F32)
    with_a = a is not None
    scratch = [
        pltpu.VMEM((NCH, C), jnp.int32),
        pltpu.VMEM((NCH, C), jnp.int32),
        pltpu.VMEM((C, F), _F32),
        pltpu.VMEM((C, F), _F32),
    ]
    if with_a:
        scratch.append(pltpu.VMEM((C, F), _F32))
    scratch += [
        pltpu.VMEM_SHARED((HALF, F), _F32),
        pltpu.SemaphoreType.DMA,
        pltpu.SemaphoreType.DMA,
    ]
    if with_a:
        scratch.append(pltpu.SemaphoreType.DMA)
    fn = pl.kernel(
        functools.partial(_conv_body, with_a),
        out_type=jax.ShapeDtypeStruct((NPAD, F), _F32),
        mesh=_mesh(),
        scratch_types=scratch,
    )
    if with_a:
        return fn(sidxT, didxT, tbl, a, zerosF)
    return fn(sidxT, didxT, tbl, zerosF)


def _mlp_chunk(eps, a_out, idx_v, rows_v, e_v, a_v, ebase, g):
    base = ebase + g * CM

    def e_body(e, cc):
        for f in range(F // 16):
            sl = pl.ds(f * 16, 16)
            em = rows_v[e, pl.ds(f * 16, 16)] * rows_v[CM + e, pl.ds(f * 16, 16)]
            es = rows_v[e, pl.ds(128 + f * 16, 16)] * rows_v[CM + e, pl.ds(128 + f * 16, 16)]
            a_v[e, sl] = em + es * e_v[e, sl]
        return cc

    lax.fori_loop(0, CM, e_body, 0)
    pltpu.sync_copy(a_v, a_out.at[pl.ds(base, CM)])


def _mlp_body(sdidx, p01_hbm, p23_hbm, eps0, eps1, a0_out, a1_out,
              idx_v, rows0_v, rows1_v, e0_v, e1_v, a_v, sem0, sem1, sem2, sem3):
    c = lax.axis_index("c")
    s = lax.axis_index("s")
    w = s * NC + c
    pltpu.sync_copy(sdidx.at[w], idx_v)
    ebase = w * EPT

    for p_hbm, eps, a_out in ((p01_hbm, eps0, a0_out), (p23_hbm, eps1, a1_out)):
        def issue(g, rows_v, sem, e_v, esem):
            pltpu.async_copy(p_hbm.at[plsc.Indices(idx_v.at[g])], rows_v, sem)
            pltpu.async_copy(eps.at[pl.ds(ebase + g * CM, CM)], e_v, esem)

        def wait(g, rows_v, sem, e_v, esem):
            pltpu.make_async_copy(
                p_hbm.at[plsc.Indices(idx_v.at[g])], rows_v, sem).wait()
            pltpu.make_async_copy(
                eps.at[pl.ds(ebase + g * CM, CM)], e_v, esem).wait()

        issue(0, rows0_v, sem0, e0_v, sem2)

        def body(g2, carry):
            g = 2 * g2
            issue(g + 1, rows1_v, sem1, e1_v, sem3)
            wait(g, rows0_v, sem0, e0_v, sem2)
            _mlp_chunk(eps, a_out, idx_v, rows0_v, e0_v, a_v, ebase, g)

            @pl.when(g + 2 < NCHM)
            def _():
                issue(g + 2, rows0_v, sem0, e0_v, sem2)

            wait(g + 1, rows1_v, sem1, e1_v, sem3)
            _mlp_chunk(eps, a_out, idx_v, rows1_v, e1_v, a_v, ebase, g + 1)
            return carry

        lax.fori_loop(0, NCHM // 2, body, 0)


def _mlp_call(sdidx, p01, p23, eps0, eps1):
    fn = pl.kernel(
        _mlp_body,
        out_type=(jax.ShapeDtypeStruct((E, F), _F32),
                  jax.ShapeDtypeStruct((E, F), _F32)),
        mesh=_mesh(),
        scratch_types=[
            pltpu.VMEM((NCHM, 2 * CM), jnp.int32),
            pltpu.VMEM((2 * CM, 2 * F), _F32),
            pltpu.VMEM((2 * CM, 2 * F), _F32),
            pltpu.VMEM((CM, F), _F32),
            pltpu.VMEM((CM, F), _F32),
            pltpu.VMEM((CM, F), _F32),
            pltpu.SemaphoreType.DMA,
            pltpu.SemaphoreType.DMA,
            pltpu.SemaphoreType.DMA,
            pltpu.SemaphoreType.DMA,
        ],
    )
    return fn(sdidx, p01, p23, eps0, eps1)


def _ni_of(degp_blk):
    return lax.rsqrt(jnp.maximum(degp_blk[0][:, :1], 1.0))


def _no_of(degp_blk):
    return lax.rsqrt(jnp.maximum(degp_blk[1][:, :1], 1.0))


_DEG_SPEC = pl.BlockSpec((2, RB, 16), lambda i: (0, i, 0))
_AGG_SPEC = pl.BlockSpec((RB, F), lambda i: (i, 0))
_ROW_SPEC = pl.BlockSpec((RB, F), lambda i: (i, 0))


def _xs_body(x_ref, degp_ref, o_ref):
    o_ref[...] = x_ref[...] * _no_of(degp_ref)


def _xs_call(x, degp):
    return pl.pallas_call(
        _xs_body,
        grid=(N // RB,),
        in_specs=[_ROW_SPEC, _DEG_SPEC],
        out_specs=_ROW_SPEC,
        out_shape=jax.ShapeDtypeStruct((N, F), _F32),
    )(x, degp)


def _node_body(agg_ref, degp_ref, w_ref, b_ref, o_ref):
    t = agg_ref[...] * _ni_of(degp_ref)
    y = jnp.dot(t, w_ref[...], preferred_element_type=_F32) + b_ref[...]
    y = jnp.maximum(y, 0.0) * _no_of(degp_ref)
    o_ref[...] = y


def _node_call(agg, degp, w, b):
    return pl.pallas_call(
        _node_body,
        grid=(N // RB,),
        in_specs=[
            _AGG_SPEC,
            _DEG_SPEC,
            pl.BlockSpec((F, F), lambda i: (0, 0)),
            pl.BlockSpec((1, F), lambda i: (0, 0)),
        ],
        out_specs=_ROW_SPEC,
        out_shape=jax.ShapeDtypeStruct((N, F), _F32),
    )(agg, degp, w, b.reshape(1, F))


def _proj_body(agg_ref, degp_ref, w1_ref, b1_ref, ws_ref, wd_ref, bc_ref,
               p01_ref, p23_ref):
    t = agg_ref[...] * _ni_of(degp_ref)
    z = jnp.dot(t, w1_ref[...], preferred_element_type=_F32) + b1_ref[...]
    z = jnp.maximum(z, 0.0)
    ps = jnp.exp(jnp.dot(z, ws_ref[...], preferred_element_type=_F32))
    pd = jnp.exp(
        jnp.dot(z, wd_ref[...], preferred_element_type=_F32) + bc_ref[...])
    p01_ref[0] = ps[:, :2 * F]
    p01_ref[1] = pd[:, :2 * F]
    p23_ref[0] = ps[:, 2 * F:]
    p23_ref[1] = pd[:, 2 * F:]


def _proj_call(agg, degp, w1, b1, ws, wd, bc):
    spec_p = pl.BlockSpec((2, RB, 2 * F), lambda i: (0, i, 0))
    return pl.pallas_call(
        _proj_body,
        grid=(N // RB,),
        in_specs=[
            _AGG_SPEC,
            _DEG_SPEC,
            pl.BlockSpec((F, F), lambda i: (0, 0)),
            pl.BlockSpec((1, F), lambda i: (0, 0)),
            pl.BlockSpec((F, 4 * F), lambda i: (0, 0)),
            pl.BlockSpec((F, 4 * F), lambda i: (0, 0)),
            pl.BlockSpec((1, 4 * F), lambda i: (0, 0)),
        ],
        out_specs=(spec_p, spec_p),
        out_shape=(jax.ShapeDtypeStruct((2, N, 2 * F), _F32),
                   jax.ShapeDtypeStruct((2, N, 2 * F), _F32)),
    )(agg, degp, w1, b1.reshape(1, F), ws, wd, bc.reshape(1, 4 * F))


def _final_body(agg_ref, degp_ref, w_ref, b_ref, o_ref):
    t = agg_ref[...] * _ni_of(degp_ref)
    y = jnp.dot(t, w_ref[...], preferred_element_type=_F32) + b_ref[...]
    m = jnp.max(y, axis=-1, keepdims=True)
    ey = jnp.exp(y - m)
    o_ref[...] = ey / jnp.sum(ey, axis=-1, keepdims=True)


def _final_call(agg, degp, w, b):
    return pl.pallas_call(
        _final_body,
        grid=(N // RB,),
        in_specs=[
            _AGG_SPEC,
            _DEG_SPEC,
            pl.BlockSpec((F, F), lambda i: (0, 0)),
            pl.BlockSpec((1, F), lambda i: (0, 0)),
        ],
        out_specs=_ROW_SPEC,
        out_shape=jax.ShapeDtypeStruct((N, F), _F32),
    )(agg, degp, w, b.reshape(1, F))


def kernel(x, edge_index, W_enc0, b_enc0, W_enc1, b_enc1,
           W_gn0, b_gn0, W_gn1, b_gn1,
           Wmu0, bmu0, Wls0, bls0, Wmu1, bmu1, Wls1, bls1,
           eps0, eps1):
    src = edge_index[0]
    dst = edge_index[1]
    sidxT = src.reshape(NS, NCH, C)
    didxT = dst.reshape(NS, NCH, C)
    sidxW = src.reshape(NW, NCHM, CM)
    didxW = dst.reshape(NW, NCHM, CM)
    sdidx = jnp.concatenate([sidxW, didxW + N], axis=-1)

    ws_cat = jnp.concatenate(
        [Wmu0[:F], Wls0[:F], Wmu1[:F], Wls1[:F]], axis=1)
    wd_cat = jnp.concatenate(
        [Wmu0[F:], Wls0[F:], Wmu1[F:], Wls1[F:]], axis=1)
    bc = jnp.concatenate([bmu0, bls0, bmu1, bls1])

    degp = _deg_call(sidxT, didxT)
    xs = _xs_call(x, degp)
    agg1 = _conv_call(sidxT, didxT, xs)
    z1s = _node_call(agg1, degp, W_enc0, b_enc0)
    agg2 = _conv_call(sidxT, didxT, z1s)
    p01, p23 = _proj_call(agg2, degp, W_enc1, b_enc1, ws_cat, wd_cat, bc)
    a0, a1 = _mlp_call(sdidx, p01.reshape(2 * N, 2 * F),
                       p23.reshape(2 * N, 2 * F), eps0, eps1)
    agg3 = _conv_call(sidxT, didxT, xs, a=a0)
    h1s = _node_call(agg3, degp, W_gn0, b_gn0)
    agg4 = _conv_call(sidxT, didxT, h1s, a=a1)
    return _final_call(agg4, degp, W_gn1, b_gn1)


# speedup vs baseline: 3.0278x; 1.1167x over previous
import functools

import jax
import jax.numpy as jnp
from jax import lax
from jax.experimental import pallas as pl
from jax.experimental.pallas import tpu as pltpu
from jax.experimental.pallas import tpu_sc as plsc

N = 10000
E = 320000
F = 128
NC = 2
NS = 16
NW = NC * NS
NPAD = 10240
HALF = NPAD // 2
RPT = HALF // NS
ETI = E // NS
C = 80
NCH = ETI // C
NSUB = 5
NCHS = NCH // NSUB
EPT = E // NW
CM = 40
NCHM = EPT // CM
RB = 2000

_F32 = jnp.float32


def _mesh():
    return plsc.VectorSubcoreMesh(core_axis_name="c", subcore_axis_name="s")


def _deg_body(sidx, didx, ones_hbm, zeros16, out,
              ones_v, oidx_v, iidx_v, degin_sh, degout_sh):
    c = lax.axis_index("c")
    s = lax.axis_index("s")
    base = c * HALF
    pltpu.sync_copy(ones_hbm, ones_v)
    pltpu.sync_copy(sidx.at[s], oidx_v)
    pltpu.sync_copy(didx.at[s], iidx_v)

    def mrow(j, carry):
        for k in range(C // 16):
            sl = pl.ds(k * 16, 16)
            ti = iidx_v[j, sl] - base
            iidx_v[j, sl] = jnp.where((ti >= 0) & (ti < HALF), ti, -1)
            to = oidx_v[j, sl] - base
            oidx_v[j, sl] = jnp.where((to >= 0) & (to < HALF), to, -1)
        return carry

    lax.fori_loop(0, NCH, mrow, 0)
    pltpu.sync_copy(zeros16, degin_sh.at[pl.ds(s * RPT, RPT)])
    pltpu.sync_copy(zeros16, degout_sh.at[pl.ds(s * RPT, RPT)])
    plsc.subcore_barrier()

    def body(j, carry):
        pltpu.sync_copy(
            ones_v, degin_sh.at[plsc.Indices(iidx_v.at[j], ignored_value=-1)],
            add=True)
        pltpu.sync_copy(
            ones_v, degout_sh.at[plsc.Indices(oidx_v.at[j], ignored_value=-1)],
            add=True)
        return carry

    lax.fori_loop(0, NCH, body, 0)
    plsc.subcore_barrier()
    pltpu.sync_copy(degin_sh.at[pl.ds(s * RPT, RPT)],
                    out.at[0, pl.ds(base + s * RPT, RPT)])
    pltpu.sync_copy(degout_sh.at[pl.ds(s * RPT, RPT)],
                    out.at[1, pl.ds(base + s * RPT, RPT)])


def _deg_call(sidxT, didxT):
    ones16 = jnp.ones((C, 16), _F32)
    zeros16 = jnp.zeros((RPT, 16), _F32)
    fn = pl.kernel(
        _deg_body,
        out_type=jax.ShapeDtypeStruct((2, NPAD, 16), _F32),
        mesh=_mesh(),
        scratch_types=[
            pltpu.VMEM((C, 16), _F32),
            pltpu.VMEM((NCH, C), jnp.int32),
            pltpu.VMEM((NCH, C), jnp.int32),
            pltpu.VMEM_SHARED((HALF, 16), _F32),
            pltpu.VMEM_SHARED((HALF, 16), _F32),
        ],
    )
    return fn(sidxT, didxT, ones16, zeros16)


def _conv_body(with_a, *refs):
    if with_a:
        (sidx, didx, tbl, a_hbm, zerosF, out,
         sidx_v, didx_v, rows0_v, rows1_v, a_v, agg_sh,
         semA, semB) = refs
    else:
        (sidx, didx, tbl, zerosF, out,
         sidx_v, didx_v, rows0_v, rows1_v, agg_sh, semA, semB) = refs
    c = lax.axis_index("c")
    s = lax.axis_index("s")
    base_n = c * HALF

    def mask_rows(nrows):
        def mrow(j, carry):
            for k in range(C // 16):
                sl = pl.ds(k * 16, 16)
                t = didx_v[j, sl] - base_n
                valid = (t >= 0) & (t < HALF)
                didx_v[j, sl] = jnp.where(valid, t, -1)
                sidx_v[j, sl] = jnp.where(valid, sidx_v[j, sl], -1)
            return carry

        lax.fori_loop(0, nrows, mrow, 0)

    def issue_rows(g, rows_v, sem):
        return pltpu.async_copy(
            tbl.at[plsc.Indices(sidx_v.at[g], ignored_value=-1)],
            rows_v, sem)

    def mul(rows_v):
        def mul_body(e, cc):
            for f in range(F // 16):
                sl = pl.ds(f * 16, 16)
                a_v[e, sl] = rows_v[e, sl] * a_v[e, sl]
            return cc

        lax.fori_loop(0, C, mul_body, 0)

    def scatter(g, src_v):
        pltpu.sync_copy(
            src_v, agg_sh.at[plsc.Indices(didx_v.at[g], ignored_value=-1)],
            add=True)

    pltpu.sync_copy(zerosF, agg_sh.at[pl.ds(s * RPT, RPT)])
    plsc.subcore_barrier()

    if with_a:
        for hh in range(NSUB):
            pltpu.sync_copy(sidx.at[s, hh], sidx_v)
            pltpu.sync_copy(didx.at[s, hh], didx_v)
            mask_rows(NCHS)
            ebase = s * ETI + hh * NCHS * C

            def body_a(g2, carry):
                g = 2 * g2
                cp0 = issue_rows(g, rows0_v, semA)
                cp1 = issue_rows(g + 1, rows1_v, semB)
                pltpu.sync_copy(a_hbm.at[pl.ds(ebase + g * C, C)], a_v)
                cp0.wait()
                mul(rows0_v)
                scatter(g, a_v)
                pltpu.sync_copy(a_hbm.at[pl.ds(ebase + (g + 1) * C, C)], a_v)
                cp1.wait()
                mul(rows1_v)
                scatter(g + 1, a_v)
                return carry

            lax.fori_loop(0, NCHS // 2, body_a, 0)
    else:
        pltpu.sync_copy(sidx.at[s], sidx_v)
        pltpu.sync_copy(didx.at[s], didx_v)
        mask_rows(NCH)

        def body(g2, carry):
            g = 2 * g2
            cp0 = issue_rows(g, rows0_v, semA)
            cp1 = issue_rows(g + 1, rows1_v, semB)
            cp0.wait()
            scatter(g, rows0_v)
            cp1.wait()
            scatter(g + 1, rows1_v)
            return carry

        lax.fori_loop(0, NCH // 2, body, 0)
    plsc.subcore_barrier()
    pltpu.sync_copy(agg_sh.at[pl.ds(s * RPT, RPT)],
                    out.at[pl.ds(c * HALF + s * RPT, RPT)])


def _conv_call(sidxT, didxT, tbl, a=None):
    zerosF = jnp.zeros((RPT, F), _F32)
    with_a = a is not None
    nidx = NCHS if with_a else NCH
    scratch = [
        pltpu.VMEM((nidx, C), jnp.int32),
        pltpu.VMEM((nidx, C), jnp.int32),
        pltpu.VMEM((C, F), _F32),
        pltpu.VMEM((C, F), _F32),
    ]
    if with_a:
        scratch.append(pltpu.VMEM((C, F), _F32))
    scratch += [
        pltpu.VMEM_SHARED((HALF, F), _F32),
        pltpu.SemaphoreType.DMA,
        pltpu.SemaphoreType.DMA,
    ]
    fn = pl.kernel(
        functools.partial(_conv_body, with_a),
        out_type=jax.ShapeDtypeStruct((NPAD, F), _F32),
        mesh=_mesh(),
        scratch_types=scratch,
    )
    if with_a:
        return fn(sidxT.reshape(NS, NSUB, NCHS, C),
                  didxT.reshape(NS, NSUB, NCHS, C), tbl, a, zerosF)
    return fn(sidxT, didxT, tbl, zerosF)


def _mlp_chunk(eps, a_out, idx_v, rows_v, e_v, a_v, ebase, g):
    base = ebase + g * CM

    def e_body(e, cc):
        for f in range(F // 16):
            sl = pl.ds(f * 16, 16)
            em = rows_v[e, pl.ds(f * 16, 16)] * rows_v[CM + e, pl.ds(f * 16, 16)]
            es = rows_v[e, pl.ds(128 + f * 16, 16)] * rows_v[CM + e, pl.ds(128 + f * 16, 16)]
            a_v[e, sl] = em + es * e_v[e, sl]
        return cc

    lax.fori_loop(0, CM, e_body, 0)
    pltpu.sync_copy(a_v, a_out.at[pl.ds(base, CM)])


def _mlp_body(sdidx, p01_hbm, p23_hbm, eps0, eps1, a0_out, a1_out,
              idx_v, rows0_v, rows1_v, e0_v, e1_v, a_v, sem0, sem1, sem2, sem3):
    c = lax.axis_index("c")
    s = lax.axis_index("s")
    w = s * NC + c
    pltpu.sync_copy(sdidx.at[w], idx_v)
    ebase = w * EPT

    for p_hbm, eps, a_out in ((p01_hbm, eps0, a0_out), (p23_hbm, eps1, a1_out)):
        def issue(g, rows_v, sem, e_v, esem):
            pltpu.async_copy(p_hbm.at[plsc.Indices(idx_v.at[g])], rows_v, sem)
            pltpu.async_copy(eps.at[pl.ds(ebase + g * CM, CM)], e_v, esem)

        def wait(g, rows_v, sem, e_v, esem):
            pltpu.make_async_copy(
                p_hbm.at[plsc.Indices(idx_v.at[g])], rows_v, sem).wait()
            pltpu.make_async_copy(
                eps.at[pl.ds(ebase + g * CM, CM)], e_v, esem).wait()

        issue(0, rows0_v, sem0, e0_v, sem2)

        def body(g2, carry):
            g = 2 * g2
            issue(g + 1, rows1_v, sem1, e1_v, sem3)
            wait(g, rows0_v, sem0, e0_v, sem2)
            _mlp_chunk(eps, a_out, idx_v, rows0_v, e0_v, a_v, ebase, g)

            @pl.when(g + 2 < NCHM)
            def _():
                issue(g + 2, rows0_v, sem0, e0_v, sem2)

            wait(g + 1, rows1_v, sem1, e1_v, sem3)
            _mlp_chunk(eps, a_out, idx_v, rows1_v, e1_v, a_v, ebase, g + 1)
            return carry

        lax.fori_loop(0, NCHM // 2, body, 0)


def _mlp_call(sdidx, p01, p23, eps0, eps1):
    fn = pl.kernel(
        _mlp_body,
        out_type=(jax.ShapeDtypeStruct((E, F), _F32),
                  jax.ShapeDtypeStruct((E, F), _F32)),
        mesh=_mesh(),
        scratch_types=[
            pltpu.VMEM((NCHM, 2 * CM), jnp.int32),
            pltpu.VMEM((2 * CM, 2 * F), _F32),
            pltpu.VMEM((2 * CM, 2 * F), _F32),
            pltpu.VMEM((CM, F), _F32),
            pltpu.VMEM((CM, F), _F32),
            pltpu.VMEM((CM, F), _F32),
            pltpu.SemaphoreType.DMA,
            pltpu.SemaphoreType.DMA,
            pltpu.SemaphoreType.DMA,
            pltpu.SemaphoreType.DMA,
        ],
    )
    return fn(sdidx, p01, p23, eps0, eps1)


def _ni_of(degp_blk):
    return lax.rsqrt(jnp.maximum(degp_blk[0][:, :1], 1.0))


def _no_of(degp_blk):
    return lax.rsqrt(jnp.maximum(degp_blk[1][:, :1], 1.0))


_DEG_SPEC = pl.BlockSpec((2, RB, 16), lambda i: (0, i, 0))
_AGG_SPEC = pl.BlockSpec((RB, F), lambda i: (i, 0))
_ROW_SPEC = pl.BlockSpec((RB, F), lambda i: (i, 0))


def _xs_body(x_ref, degp_ref, o_ref):
    o_ref[...] = x_ref[...] * _no_of(degp_ref)


def _xs_call(x, degp):
    return pl.pallas_call(
        _xs_body,
        grid=(N // RB,),
        in_specs=[_ROW_SPEC, _DEG_SPEC],
        out_specs=_ROW_SPEC,
        out_shape=jax.ShapeDtypeStruct((N, F), _F32),
    )(x, degp)


def _node_body(agg_ref, degp_ref, w_ref, b_ref, o_ref):
    t = agg_ref[...] * _ni_of(degp_ref)
    y = jnp.dot(t, w_ref[...], preferred_element_type=_F32) + b_ref[...]
    y = jnp.maximum(y, 0.0) * _no_of(degp_ref)
    o_ref[...] = y


def _node_call(agg, degp, w, b):
    return pl.pallas_call(
        _node_body,
        grid=(N // RB,),
        in_specs=[
            _AGG_SPEC,
            _DEG_SPEC,
            pl.BlockSpec((F, F), lambda i: (0, 0)),
            pl.BlockSpec((1, F), lambda i: (0, 0)),
        ],
        out_specs=_ROW_SPEC,
        out_shape=jax.ShapeDtypeStruct((N, F), _F32),
    )(agg, degp, w, b.reshape(1, F))


def _proj_body(agg_ref, degp_ref, w1_ref, b1_ref, ws_ref, wd_ref, bc_ref,
               p01_ref, p23_ref):
    t = agg_ref[...] * _ni_of(degp_ref)
    z = jnp.dot(t, w1_ref[...], preferred_element_type=_F32) + b1_ref[...]
    z = jnp.maximum(z, 0.0)
    ps = jnp.exp(jnp.dot(z, ws_ref[...], preferred_element_type=_F32))
    pd = jnp.exp(
        jnp.dot(z, wd_ref[...], preferred_element_type=_F32) + bc_ref[...])
    p01_ref[0] = ps[:, :2 * F]
    p01_ref[1] = pd[:, :2 * F]
    p23_ref[0] = ps[:, 2 * F:]
    p23_ref[1] = pd[:, 2 * F:]


def _proj_call(agg, degp, w1, b1, ws, wd, bc):
    spec_p = pl.BlockSpec((2, RB, 2 * F), lambda i: (0, i, 0))
    return pl.pallas_call(
        _proj_body,
        grid=(N // RB,),
        in_specs=[
            _AGG_SPEC,
            _DEG_SPEC,
            pl.BlockSpec((F, F), lambda i: (0, 0)),
            pl.BlockSpec((1, F), lambda i: (0, 0)),
            pl.BlockSpec((F, 4 * F), lambda i: (0, 0)),
            pl.BlockSpec((F, 4 * F), lambda i: (0, 0)),
            pl.BlockSpec((1, 4 * F), lambda i: (0, 0)),
        ],
        out_specs=(spec_p, spec_p),
        out_shape=(jax.ShapeDtypeStruct((2, N, 2 * F), _F32),
                   jax.ShapeDtypeStruct((2, N, 2 * F), _F32)),
    )(agg, degp, w1, b1.reshape(1, F), ws, wd, bc.reshape(1, 4 * F))


def _final_body(agg_ref, degp_ref, w_ref, b_ref, o_ref):
    t = agg_ref[...] * _ni_of(degp_ref)
    y = jnp.dot(t, w_ref[...], preferred_element_type=_F32) + b_ref[...]
    m = jnp.max(y, axis=-1, keepdims=True)
    ey = jnp.exp(y - m)
    o_ref[...] = ey / jnp.sum(ey, axis=-1, keepdims=True)


def _final_call(agg, degp, w, b):
    return pl.pallas_call(
        _final_body,
        grid=(N // RB,),
        in_specs=[
            _AGG_SPEC,
            _DEG_SPEC,
            pl.BlockSpec((F, F), lambda i: (0, 0)),
            pl.BlockSpec((1, F), lambda i: (0, 0)),
        ],
        out_specs=_ROW_SPEC,
        out_shape=jax.ShapeDtypeStruct((N, F), _F32),
    )(agg, degp, w, b.reshape(1, F))


def kernel(x, edge_index, W_enc0, b_enc0, W_enc1, b_enc1,
           W_gn0, b_gn0, W_gn1, b_gn1,
           Wmu0, bmu0, Wls0, bls0, Wmu1, bmu1, Wls1, bls1,
           eps0, eps1):
    src = edge_index[0]
    dst = edge_index[1]
    sidxT = src.reshape(NS, NCH, C)
    didxT = dst.reshape(NS, NCH, C)
    sidxW = src.reshape(NW, NCHM, CM)
    didxW = dst.reshape(NW, NCHM, CM)
    sdidx = jnp.concatenate([sidxW, didxW + N], axis=-1)

    ws_cat = jnp.concatenate(
        [Wmu0[:F], Wls0[:F], Wmu1[:F], Wls1[:F]], axis=1)
    wd_cat = jnp.concatenate(
        [Wmu0[F:], Wls0[F:], Wmu1[F:], Wls1[F:]], axis=1)
    bc = jnp.concatenate([bmu0, bls0, bmu1, bls1])

    degp = _deg_call(sidxT, didxT)
    xs = _xs_call(x, degp)
    agg1 = _conv_call(sidxT, didxT, xs)
    z1s = _node_call(agg1, degp, W_enc0, b_enc0)
    agg2 = _conv_call(sidxT, didxT, z1s)
    p01, p23 = _proj_call(agg2, degp, W_enc1, b_enc1, ws_cat, wd_cat, bc)
    a0, a1 = _mlp_call(sdidx, p01.reshape(2 * N, 2 * F),
                       p23.reshape(2 * N, 2 * F), eps0, eps1)
    agg3 = _conv_call(sidxT, didxT, xs, a=a0)
    h1s = _node_call(agg3, degp, W_gn0, b_gn0)
    agg4 = _conv_call(sidxT, didxT, h1s, a=a1)
    return _final_call(agg4, degp, W_gn1, b_gn1)


# speedup vs baseline: 3.1414x; 1.0375x over previous
import functools

import jax
import jax.numpy as jnp
from jax import lax
from jax.experimental import pallas as pl
from jax.experimental.pallas import tpu as pltpu
from jax.experimental.pallas import tpu_sc as plsc

N = 10000
E = 320000
F = 128
NC = 2
NS = 16
NW = NC * NS
NPAD = 10240
HALF = NPAD // 2
RPT = HALF // NS
ETI = E // NS
C = 80
NCH = ETI // C
NSUB = 5
NCHS = NCH // NSUB
EPT = E // NW
CM = 40
NCHM = EPT // CM
RB = 2000

_F32 = jnp.float32


def _mesh():
    return plsc.VectorSubcoreMesh(core_axis_name="c", subcore_axis_name="s")


def _deg_body(sidx, didx, ones_hbm, zeros16, out,
              ones_v, oidx_v, iidx_v, degin_sh, degout_sh):
    c = lax.axis_index("c")
    s = lax.axis_index("s")
    base = c * HALF
    pltpu.sync_copy(ones_hbm, ones_v)
    pltpu.sync_copy(sidx.at[s], oidx_v)
    pltpu.sync_copy(didx.at[s], iidx_v)

    def mrow(j, carry):
        for k in range(C // 16):
            sl = pl.ds(k * 16, 16)
            ti = iidx_v[j, sl] - base
            iidx_v[j, sl] = jnp.where((ti >= 0) & (ti < HALF), ti, -1)
            to = oidx_v[j, sl] - base
            oidx_v[j, sl] = jnp.where((to >= 0) & (to < HALF), to, -1)
        return carry

    lax.fori_loop(0, NCH, mrow, 0)
    pltpu.sync_copy(zeros16, degin_sh.at[pl.ds(s * RPT, RPT)])
    pltpu.sync_copy(zeros16, degout_sh.at[pl.ds(s * RPT, RPT)])
    plsc.subcore_barrier()

    def body(j, carry):
        pltpu.sync_copy(
            ones_v, degin_sh.at[plsc.Indices(iidx_v.at[j], ignored_value=-1)],
            add=True)
        pltpu.sync_copy(
            ones_v, degout_sh.at[plsc.Indices(oidx_v.at[j], ignored_value=-1)],
            add=True)
        return carry

    lax.fori_loop(0, NCH, body, 0)
    plsc.subcore_barrier()
    pltpu.sync_copy(degin_sh.at[pl.ds(s * RPT, RPT)],
                    out.at[0, pl.ds(base + s * RPT, RPT)])
    pltpu.sync_copy(degout_sh.at[pl.ds(s * RPT, RPT)],
                    out.at[1, pl.ds(base + s * RPT, RPT)])


def _deg_call(sidxT, didxT):
    ones16 = jnp.ones((C, 16), _F32)
    zeros16 = jnp.zeros((RPT, 16), _F32)
    fn = pl.kernel(
        _deg_body,
        out_type=jax.ShapeDtypeStruct((2, NPAD, 16), _F32),
        mesh=_mesh(),
        scratch_types=[
            pltpu.VMEM((C, 16), _F32),
            pltpu.VMEM((NCH, C), jnp.int32),
            pltpu.VMEM((NCH, C), jnp.int32),
            pltpu.VMEM_SHARED((HALF, 16), _F32),
            pltpu.VMEM_SHARED((HALF, 16), _F32),
        ],
    )
    return fn(sidxT, didxT, ones16, zeros16)


def _conv_body(with_a, *refs):
    if with_a:
        (sidx, didx, tbl, a_hbm, zerosF, out,
         sidx_v, didx_v, rows0_v, rows1_v, a_v, agg_sh,
         semA, semB) = refs
    else:
        (sidx, didx, tbl, zerosF, out,
         sidx_v, didx_v, rows0_v, rows1_v, agg_sh, semA, semB) = refs
    c = lax.axis_index("c")
    s = lax.axis_index("s")
    base_n = c * HALF

    def mask_rows(nrows):
        def mrow(j, carry):
            for k in range(C // 16):
                sl = pl.ds(k * 16, 16)
                t = didx_v[j, sl] - base_n
                valid = (t >= 0) & (t < HALF)
                didx_v[j, sl] = jnp.where(valid, t, -1)
                sidx_v[j, sl] = jnp.where(valid, sidx_v[j, sl], -1)
            return carry

        lax.fori_loop(0, nrows, mrow, 0)

    def issue_rows(g, rows_v, sem):
        return pltpu.async_copy(
            tbl.at[plsc.Indices(sidx_v.at[g], ignored_value=-1)],
            rows_v, sem)

    def mul(rows_v):
        def mul_body(e, cc):
            for f in range(F // 16):
                sl = pl.ds(f * 16, 16)
                a_v[e, sl] = rows_v[e, sl] * a_v[e, sl]
            return cc

        lax.fori_loop(0, C, mul_body, 0)

    def scatter(g, src_v):
        pltpu.sync_copy(
            src_v, agg_sh.at[plsc.Indices(didx_v.at[g], ignored_value=-1)],
            add=True)

    pltpu.sync_copy(zerosF, agg_sh.at[pl.ds(s * RPT, RPT)])
    plsc.subcore_barrier()

    if with_a:
        for hh in range(NSUB):
            pltpu.sync_copy(sidx.at[s, hh], sidx_v)
            pltpu.sync_copy(didx.at[s, hh], didx_v)
            mask_rows(NCHS)
            ebase = s * ETI + hh * NCHS * C

            def body_a(g2, carry):
                g = 2 * g2
                cp0 = issue_rows(g, rows0_v, semA)
                cp1 = issue_rows(g + 1, rows1_v, semB)
                pltpu.sync_copy(a_hbm.at[pl.ds(ebase + g * C, C)], a_v)
                cp0.wait()
                mul(rows0_v)
                scatter(g, a_v)
                pltpu.sync_copy(a_hbm.at[pl.ds(ebase + (g + 1) * C, C)], a_v)
                cp1.wait()
                mul(rows1_v)
                scatter(g + 1, a_v)
                return carry

            lax.fori_loop(0, NCHS // 2, body_a, 0)
    else:
        pltpu.sync_copy(sidx.at[s], sidx_v)
        pltpu.sync_copy(didx.at[s], didx_v)
        mask_rows(NCH)

        def body(g2, carry):
            g = 2 * g2
            cp0 = issue_rows(g, rows0_v, semA)
            cp1 = issue_rows(g + 1, rows1_v, semB)
            cp0.wait()
            scatter(g, rows0_v)
            cp1.wait()
            scatter(g + 1, rows1_v)
            return carry

        lax.fori_loop(0, NCH // 2, body, 0)
    plsc.subcore_barrier()
    pltpu.sync_copy(agg_sh.at[pl.ds(s * RPT, RPT)],
                    out.at[pl.ds(c * HALF + s * RPT, RPT)])


def _conv_call(sidxT, didxT, tbl, a=None):
    zerosF = jnp.zeros((RPT, F), _F32)
    with_a = a is not None
    nidx = NCHS if with_a else NCH
    scratch = [
        pltpu.VMEM((nidx, C), jnp.int32),
        pltpu.VMEM((nidx, C), jnp.int32),
        pltpu.VMEM((C, F), _F32),
        pltpu.VMEM((C, F), _F32),
    ]
    if with_a:
        scratch.append(pltpu.VMEM((C, F), _F32))
    scratch += [
        pltpu.VMEM_SHARED((HALF, F), _F32),
        pltpu.SemaphoreType.DMA,
        pltpu.SemaphoreType.DMA,
    ]
    fn = pl.kernel(
        functools.partial(_conv_body, with_a),
        out_type=jax.ShapeDtypeStruct((NPAD, F), _F32),
        mesh=_mesh(),
        scratch_types=scratch,
    )
    if with_a:
        return fn(sidxT.reshape(NS, NSUB, NCHS, C),
                  didxT.reshape(NS, NSUB, NCHS, C), tbl, a, zerosF)
    return fn(sidxT, didxT, tbl, zerosF)


def _mlp_chunk(eps, a_out, idx_v, rows_v, e_v, a_v, ebase, g, ssem):
    base = ebase + g * CM

    @pl.when(g >= 2)
    def _():
        pltpu.make_async_copy(
            a_v, a_out.at[pl.ds(ebase + (g - 2) * CM, CM)], ssem).wait()

    def e_body(e, cc):
        for f in range(F // 16):
            sl = pl.ds(f * 16, 16)
            em = rows_v[e, pl.ds(f * 16, 16)] * rows_v[CM + e, pl.ds(f * 16, 16)]
            es = rows_v[e, pl.ds(128 + f * 16, 16)] * rows_v[CM + e, pl.ds(128 + f * 16, 16)]
            a_v[e, sl] = em + es * e_v[e, sl]
        return cc

    lax.fori_loop(0, CM, e_body, 0)
    pltpu.async_copy(a_v, a_out.at[pl.ds(base, CM)], ssem)


def _mlp_body(sdidx, p01_hbm, p23_hbm, eps0, eps1, a0_out, a1_out,
              idx_v, rows0_v, rows1_v, e0_v, e1_v, a0_v, a1_v,
              sem0, sem1, sem2, sem3, ssem0, ssem1):
    c = lax.axis_index("c")
    s = lax.axis_index("s")
    w = s * NC + c
    pltpu.sync_copy(sdidx.at[w], idx_v)
    ebase = w * EPT

    for p_hbm, eps, a_out in ((p01_hbm, eps0, a0_out), (p23_hbm, eps1, a1_out)):
        def issue(g, rows_v, sem, e_v, esem):
            pltpu.async_copy(p_hbm.at[plsc.Indices(idx_v.at[g])], rows_v, sem)
            pltpu.async_copy(eps.at[pl.ds(ebase + g * CM, CM)], e_v, esem)

        def wait(g, rows_v, sem, e_v, esem):
            pltpu.make_async_copy(
                p_hbm.at[plsc.Indices(idx_v.at[g])], rows_v, sem).wait()
            pltpu.make_async_copy(
                eps.at[pl.ds(ebase + g * CM, CM)], e_v, esem).wait()

        issue(0, rows0_v, sem0, e0_v, sem2)

        def body(g2, carry):
            g = 2 * g2
            issue(g + 1, rows1_v, sem1, e1_v, sem3)
            wait(g, rows0_v, sem0, e0_v, sem2)
            _mlp_chunk(eps, a_out, idx_v, rows0_v, e0_v, a0_v, ebase, g, ssem0)

            @pl.when(g + 2 < NCHM)
            def _():
                issue(g + 2, rows0_v, sem0, e0_v, sem2)

            wait(g + 1, rows1_v, sem1, e1_v, sem3)
            _mlp_chunk(eps, a_out, idx_v, rows1_v, e1_v, a1_v, ebase, g + 1,
                       ssem1)
            return carry

        lax.fori_loop(0, NCHM // 2, body, 0)
        pltpu.make_async_copy(
            a0_v, a_out.at[pl.ds(ebase + (NCHM - 2) * CM, CM)], ssem0).wait()
        pltpu.make_async_copy(
            a1_v, a_out.at[pl.ds(ebase + (NCHM - 1) * CM, CM)], ssem1).wait()


def _mlp_call(sdidx, p01, p23, eps0, eps1):
    fn = pl.kernel(
        _mlp_body,
        out_type=(jax.ShapeDtypeStruct((E, F), _F32),
                  jax.ShapeDtypeStruct((E, F), _F32)),
        mesh=_mesh(),
        scratch_types=[
            pltpu.VMEM((NCHM, 2 * CM), jnp.int32),
            pltpu.VMEM((2 * CM, 2 * F), _F32),
            pltpu.VMEM((2 * CM, 2 * F), _F32),
            pltpu.VMEM((CM, F), _F32),
            pltpu.VMEM((CM, F), _F32),
            pltpu.VMEM((CM, F), _F32),
            pltpu.VMEM((CM, F), _F32),
            pltpu.SemaphoreType.DMA,
            pltpu.SemaphoreType.DMA,
            pltpu.SemaphoreType.DMA,
            pltpu.SemaphoreType.DMA,
            pltpu.SemaphoreType.DMA,
            pltpu.SemaphoreType.DMA,
        ],
    )
    return fn(sdidx, p01, p23, eps0, eps1)


def _ni_of(degp_blk):
    return lax.rsqrt(jnp.maximum(degp_blk[0][:, :1], 1.0))


def _no_of(degp_blk):
    return lax.rsqrt(jnp.maximum(degp_blk[1][:, :1], 1.0))


_DEG_SPEC = pl.BlockSpec((2, RB, 16), lambda i: (0, i, 0))
_AGG_SPEC = pl.BlockSpec((RB, F), lambda i: (i, 0))
_ROW_SPEC = pl.BlockSpec((RB, F), lambda i: (i, 0))


def _xs_body(x_ref, degp_ref, o_ref):
    o_ref[...] = x_ref[...] * _no_of(degp_ref)


def _xs_call(x, degp):
    return pl.pallas_call(
        _xs_body,
        grid=(N // RB,),
        in_specs=[_ROW_SPEC, _DEG_SPEC],
        out_specs=_ROW_SPEC,
        out_shape=jax.ShapeDtypeStruct((N, F), _F32),
    )(x, degp)


def _node_body(agg_ref, degp_ref, w_ref, b_ref, o_ref):
    t = agg_ref[...] * _ni_of(degp_ref)
    y = jnp.dot(t, w_ref[...], preferred_element_type=_F32) + b_ref[...]
    y = jnp.maximum(y, 0.0) * _no_of(degp_ref)
    o_ref[...] = y


def _node_call(agg, degp, w, b):
    return pl.pallas_call(
        _node_body,
        grid=(N // RB,),
        in_specs=[
            _AGG_SPEC,
            _DEG_SPEC,
            pl.BlockSpec((F, F), lambda i: (0, 0)),
            pl.BlockSpec((1, F), lambda i: (0, 0)),
        ],
        out_specs=_ROW_SPEC,
        out_shape=jax.ShapeDtypeStruct((N, F), _F32),
    )(agg, degp, w, b.reshape(1, F))


def _proj_body(agg_ref, degp_ref, w1_ref, b1_ref, ws_ref, wd_ref, bc_ref,
               p01_ref, p23_ref):
    t = agg_ref[...] * _ni_of(degp_ref)
    z = jnp.dot(t, w1_ref[...], preferred_element_type=_F32) + b1_ref[...]
    z = jnp.maximum(z, 0.0)
    ps = jnp.exp(jnp.dot(z, ws_ref[...], preferred_element_type=_F32))
    pd = jnp.exp(
        jnp.dot(z, wd_ref[...], preferred_element_type=_F32) + bc_ref[...])
    p01_ref[0] = ps[:, :2 * F]
    p01_ref[1] = pd[:, :2 * F]
    p23_ref[0] = ps[:, 2 * F:]
    p23_ref[1] = pd[:, 2 * F:]


def _proj_call(agg, degp, w1, b1, ws, wd, bc):
    spec_p = pl.BlockSpec((2, RB, 2 * F), lambda i: (0, i, 0))
    return pl.pallas_call(
        _proj_body,
        grid=(N // RB,),
        in_specs=[
            _AGG_SPEC,
            _DEG_SPEC,
            pl.BlockSpec((F, F), lambda i: (0, 0)),
            pl.BlockSpec((1, F), lambda i: (0, 0)),
            pl.BlockSpec((F, 4 * F), lambda i: (0, 0)),
            pl.BlockSpec((F, 4 * F), lambda i: (0, 0)),
            pl.BlockSpec((1, 4 * F), lambda i: (0, 0)),
        ],
        out_specs=(spec_p, spec_p),
        out_shape=(jax.ShapeDtypeStruct((2, N, 2 * F), _F32),
                   jax.ShapeDtypeStruct((2, N, 2 * F), _F32)),
    )(agg, degp, w1, b1.reshape(1, F), ws, wd, bc.reshape(1, 4 * F))


def _final_body(agg_ref, degp_ref, w_ref, b_ref, o_ref):
    t = agg_ref[...] * _ni_of(degp_ref)
    y = jnp.dot(t, w_ref[...], preferred_element_type=_F32) + b_ref[...]
    m = jnp.max(y, axis=-1, keepdims=True)
    ey = jnp.exp(y - m)
    o_ref[...] = ey / jnp.sum(ey, axis=-1, keepdims=True)


def _final_call(agg, degp, w, b):
    return pl.pallas_call(
        _final_body,
        grid=(N // RB,),
        in_specs=[
            _AGG_SPEC,
            _DEG_SPEC,
            pl.BlockSpec((F, F), lambda i: (0, 0)),
            pl.BlockSpec((1, F), lambda i: (0, 0)),
        ],
        out_specs=_ROW_SPEC,
        out_shape=jax.ShapeDtypeStruct((N, F), _F32),
    )(agg, degp, w, b.reshape(1, F))


def kernel(x, edge_index, W_enc0, b_enc0, W_enc1, b_enc1,
           W_gn0, b_gn0, W_gn1, b_gn1,
           Wmu0, bmu0, Wls0, bls0, Wmu1, bmu1, Wls1, bls1,
           eps0, eps1):
    src = edge_index[0]
    dst = edge_index[1]
    sidxT = src.reshape(NS, NCH, C)
    didxT = dst.reshape(NS, NCH, C)
    sidxW = src.reshape(NW, NCHM, CM)
    didxW = dst.reshape(NW, NCHM, CM)
    sdidx = jnp.concatenate([sidxW, didxW + N], axis=-1)

    ws_cat = jnp.concatenate(
        [Wmu0[:F], Wls0[:F], Wmu1[:F], Wls1[:F]], axis=1)
    wd_cat = jnp.concatenate(
        [Wmu0[F:], Wls0[F:], Wmu1[F:], Wls1[F:]], axis=1)
    bc = jnp.concatenate([bmu0, bls0, bmu1, bls1])

    degp = _deg_call(sidxT, didxT)
    xs = _xs_call(x, degp)
    agg1 = _conv_call(sidxT, didxT, xs)
    z1s = _node_call(agg1, degp, W_enc0, b_enc0)
    agg2 = _conv_call(sidxT, didxT, z1s)
    p01, p23 = _proj_call(agg2, degp, W_enc1, b_enc1, ws_cat, wd_cat, bc)
    a0, a1 = _mlp_call(sdidx, p01.reshape(2 * N, 2 * F),
                       p23.reshape(2 * N, 2 * F), eps0, eps1)
    agg3 = _conv_call(sidxT, didxT, xs, a=a0)
    h1s = _node_call(agg3, degp, W_gn0, b_gn0)
    agg4 = _conv_call(sidxT, didxT, h1s, a=a1)
    return _final_call(agg4, degp, W_gn1, b_gn1)


# speedup vs baseline: 3.2963x; 1.0493x over previous
import functools

import jax
import jax.numpy as jnp
from jax import lax
from jax.experimental import pallas as pl
from jax.experimental.pallas import tpu as pltpu
from jax.experimental.pallas import tpu_sc as plsc

N = 10000
E = 320000
F = 128
NC = 2
NS = 16
NW = NC * NS
NPAD = 10240
HALF = NPAD // 2
RPT = HALF // NS
ETI = E // NS
C = 80
NCH = ETI // C
NSUB = 5
NCHS = NCH // NSUB
EPT = E // NW
CM = 40
NCHM = EPT // CM
RB = 2000

_F32 = jnp.float32


def _mesh():
    return plsc.VectorSubcoreMesh(core_axis_name="c", subcore_axis_name="s")


def _deg_body(sidx, didx, ones_hbm, zeros16, out,
              ones_v, oidx_v, iidx_v, degin_sh, degout_sh):
    c = lax.axis_index("c")
    s = lax.axis_index("s")
    base = c * HALF
    pltpu.sync_copy(ones_hbm, ones_v)
    pltpu.sync_copy(sidx.at[s], oidx_v)
    pltpu.sync_copy(didx.at[s], iidx_v)

    def mrow(j, carry):
        for k in range(C // 16):
            sl = pl.ds(k * 16, 16)
            ti = iidx_v[j, sl] - base
            iidx_v[j, sl] = jnp.where((ti >= 0) & (ti < HALF), ti, -1)
            to = oidx_v[j, sl] - base
            oidx_v[j, sl] = jnp.where((to >= 0) & (to < HALF), to, -1)
        return carry

    lax.fori_loop(0, NCH, mrow, 0)
    pltpu.sync_copy(zeros16, degin_sh.at[pl.ds(s * RPT, RPT)])
    pltpu.sync_copy(zeros16, degout_sh.at[pl.ds(s * RPT, RPT)])
    plsc.subcore_barrier()

    def body(j, carry):
        pltpu.sync_copy(
            ones_v, degin_sh.at[plsc.Indices(iidx_v.at[j], ignored_value=-1)],
            add=True)
        pltpu.sync_copy(
            ones_v, degout_sh.at[plsc.Indices(oidx_v.at[j], ignored_value=-1)],
            add=True)
        return carry

    lax.fori_loop(0, NCH, body, 0)
    plsc.subcore_barrier()
    pltpu.sync_copy(degin_sh.at[pl.ds(s * RPT, RPT)],
                    out.at[0, pl.ds(base + s * RPT, RPT)])
    pltpu.sync_copy(degout_sh.at[pl.ds(s * RPT, RPT)],
                    out.at[1, pl.ds(base + s * RPT, RPT)])


def _deg_call(sidxT, didxT):
    ones16 = jnp.ones((C, 16), _F32)
    zeros16 = jnp.zeros((RPT, 16), _F32)
    fn = pl.kernel(
        _deg_body,
        out_type=jax.ShapeDtypeStruct((2, NPAD, 16), _F32),
        mesh=_mesh(),
        scratch_types=[
            pltpu.VMEM((C, 16), _F32),
            pltpu.VMEM((NCH, C), jnp.int32),
            pltpu.VMEM((NCH, C), jnp.int32),
            pltpu.VMEM_SHARED((HALF, 16), _F32),
            pltpu.VMEM_SHARED((HALF, 16), _F32),
        ],
    )
    return fn(sidxT, didxT, ones16, zeros16)


def _conv_body(with_a, *refs):
    if with_a:
        (sidx, didx, tbl, a_hbm, zerosF, out,
         sidx_v, didx_v, rows0_v, rows1_v, a0_v, a1_v, agg_sh,
         semA, semB, semC, semD) = refs
    else:
        (sidx, didx, tbl, zerosF, out,
         sidx_v, didx_v, rows0_v, rows1_v, agg_sh, semA, semB) = refs
    c = lax.axis_index("c")
    s = lax.axis_index("s")
    base_n = c * HALF

    def mask_rows(nrows):
        def mrow(j, carry):
            for k in range(C // 16):
                sl = pl.ds(k * 16, 16)
                t = didx_v[j, sl] - base_n
                valid = (t >= 0) & (t < HALF)
                didx_v[j, sl] = jnp.where(valid, t, -1)
                sidx_v[j, sl] = jnp.where(valid, sidx_v[j, sl], -1)
            return carry

        lax.fori_loop(0, nrows, mrow, 0)

    def issue_rows(g, rows_v, sem):
        return pltpu.async_copy(
            tbl.at[plsc.Indices(sidx_v.at[g], ignored_value=-1)],
            rows_v, sem)

    def mul(rows_v, a_v):
        def mul_body(e, cc):
            for f in range(F // 16):
                sl = pl.ds(f * 16, 16)
                a_v[e, sl] = rows_v[e, sl] * a_v[e, sl]
            return cc

        lax.fori_loop(0, C, mul_body, 0)

    def scatter(g, src_v):
        pltpu.sync_copy(
            src_v, agg_sh.at[plsc.Indices(didx_v.at[g], ignored_value=-1)],
            add=True)

    pltpu.sync_copy(zerosF, agg_sh.at[pl.ds(s * RPT, RPT)])
    plsc.subcore_barrier()

    if with_a:
        for hh in range(NSUB):
            pltpu.sync_copy(sidx.at[s, hh], sidx_v)
            pltpu.sync_copy(didx.at[s, hh], didx_v)
            mask_rows(NCHS)
            ebase = s * ETI + hh * NCHS * C

            def body_a(g2, carry):
                g = 2 * g2
                cp0 = issue_rows(g, rows0_v, semA)
                cp1 = issue_rows(g + 1, rows1_v, semB)
                ca0 = pltpu.async_copy(
                    a_hbm.at[pl.ds(ebase + g * C, C)], a0_v, semC)
                ca1 = pltpu.async_copy(
                    a_hbm.at[pl.ds(ebase + (g + 1) * C, C)], a1_v, semD)
                cp0.wait()
                ca0.wait()
                mul(rows0_v, a0_v)
                scatter(g, a0_v)
                cp1.wait()
                ca1.wait()
                mul(rows1_v, a1_v)
                scatter(g + 1, a1_v)
                return carry

            lax.fori_loop(0, NCHS // 2, body_a, 0)
    else:
        pltpu.sync_copy(sidx.at[s], sidx_v)
        pltpu.sync_copy(didx.at[s], didx_v)
        mask_rows(NCH)

        def body(g2, carry):
            g = 2 * g2
            cp0 = issue_rows(g, rows0_v, semA)
            cp1 = issue_rows(g + 1, rows1_v, semB)
            cp0.wait()
            scatter(g, rows0_v)
            cp1.wait()
            scatter(g + 1, rows1_v)
            return carry

        lax.fori_loop(0, NCH // 2, body, 0)
    plsc.subcore_barrier()
    pltpu.sync_copy(agg_sh.at[pl.ds(s * RPT, RPT)],
                    out.at[pl.ds(c * HALF + s * RPT, RPT)])


def _conv_call(sidxT, didxT, tbl, a=None):
    zerosF = jnp.zeros((RPT, F), _F32)
    with_a = a is not None
    nidx = NCHS if with_a else NCH
    scratch = [
        pltpu.VMEM((nidx, C), jnp.int32),
        pltpu.VMEM((nidx, C), jnp.int32),
        pltpu.VMEM((C, F), _F32),
        pltpu.VMEM((C, F), _F32),
    ]
    if with_a:
        scratch += [pltpu.VMEM((C, F), _F32), pltpu.VMEM((C, F), _F32)]
    scratch += [
        pltpu.VMEM_SHARED((HALF, F), _F32),
        pltpu.SemaphoreType.DMA,
        pltpu.SemaphoreType.DMA,
    ]
    if with_a:
        scratch += [pltpu.SemaphoreType.DMA, pltpu.SemaphoreType.DMA]
    fn = pl.kernel(
        functools.partial(_conv_body, with_a),
        out_type=jax.ShapeDtypeStruct((NPAD, F), _F32),
        mesh=_mesh(),
        scratch_types=scratch,
    )
    if with_a:
        return fn(sidxT.reshape(NS, NSUB, NCHS, C),
                  didxT.reshape(NS, NSUB, NCHS, C), tbl, a, zerosF)
    return fn(sidxT, didxT, tbl, zerosF)


def _mlp_chunk(eps, a_out, idx_v, rows_v, e_v, a_v, ebase, g, ssem):
    base = ebase + g * CM

    @pl.when(g >= 2)
    def _():
        pltpu.make_async_copy(
            a_v, a_out.at[pl.ds(ebase + (g - 2) * CM, CM)], ssem).wait()

    def e_body(e, cc):
        for f in range(F // 16):
            sl = pl.ds(f * 16, 16)
            em = rows_v[e, pl.ds(f * 16, 16)] * rows_v[CM + e, pl.ds(f * 16, 16)]
            es = rows_v[e, pl.ds(128 + f * 16, 16)] * rows_v[CM + e, pl.ds(128 + f * 16, 16)]
            a_v[e, sl] = em + es * e_v[e, sl]
        return cc

    lax.fori_loop(0, CM, e_body, 0)
    pltpu.async_copy(a_v, a_out.at[pl.ds(base, CM)], ssem)


def _mlp_body(sdidx, p01_hbm, p23_hbm, eps0, eps1, a0_out, a1_out,
              idx_v, rows0_v, rows1_v, e0_v, e1_v, a0_v, a1_v,
              sem0, sem1, sem2, sem3, ssem0, ssem1):
    c = lax.axis_index("c")
    s = lax.axis_index("s")
    w = s * NC + c
    pltpu.sync_copy(sdidx.at[w], idx_v)
    ebase = w * EPT

    for p_hbm, eps, a_out in ((p01_hbm, eps0, a0_out), (p23_hbm, eps1, a1_out)):
        def issue(g, rows_v, sem, e_v, esem):
            pltpu.async_copy(p_hbm.at[plsc.Indices(idx_v.at[g])], rows_v, sem)
            pltpu.async_copy(eps.at[pl.ds(ebase + g * CM, CM)], e_v, esem)

        def wait(g, rows_v, sem, e_v, esem):
            pltpu.make_async_copy(
                p_hbm.at[plsc.Indices(idx_v.at[g])], rows_v, sem).wait()
            pltpu.make_async_copy(
                eps.at[pl.ds(ebase + g * CM, CM)], e_v, esem).wait()

        issue(0, rows0_v, sem0, e0_v, sem2)

        def body(g2, carry):
            g = 2 * g2
            issue(g + 1, rows1_v, sem1, e1_v, sem3)
            wait(g, rows0_v, sem0, e0_v, sem2)
            _mlp_chunk(eps, a_out, idx_v, rows0_v, e0_v, a0_v, ebase, g, ssem0)

            @pl.when(g + 2 < NCHM)
            def _():
                issue(g + 2, rows0_v, sem0, e0_v, sem2)

            wait(g + 1, rows1_v, sem1, e1_v, sem3)
            _mlp_chunk(eps, a_out, idx_v, rows1_v, e1_v, a1_v, ebase, g + 1,
                       ssem1)
            return carry

        lax.fori_loop(0, NCHM // 2, body, 0)
        pltpu.make_async_copy(
            a0_v, a_out.at[pl.ds(ebase + (NCHM - 2) * CM, CM)], ssem0).wait()
        pltpu.make_async_copy(
            a1_v, a_out.at[pl.ds(ebase + (NCHM - 1) * CM, CM)], ssem1).wait()


def _mlp_call(sdidx, p01, p23, eps0, eps1):
    fn = pl.kernel(
        _mlp_body,
        out_type=(jax.ShapeDtypeStruct((E, F), _F32),
                  jax.ShapeDtypeStruct((E, F), _F32)),
        mesh=_mesh(),
        scratch_types=[
            pltpu.VMEM((NCHM, 2 * CM), jnp.int32),
            pltpu.VMEM((2 * CM, 2 * F), _F32),
            pltpu.VMEM((2 * CM, 2 * F), _F32),
            pltpu.VMEM((CM, F), _F32),
            pltpu.VMEM((CM, F), _F32),
            pltpu.VMEM((CM, F), _F32),
            pltpu.VMEM((CM, F), _F32),
            pltpu.SemaphoreType.DMA,
            pltpu.SemaphoreType.DMA,
            pltpu.SemaphoreType.DMA,
            pltpu.SemaphoreType.DMA,
            pltpu.SemaphoreType.DMA,
            pltpu.SemaphoreType.DMA,
        ],
    )
    return fn(sdidx, p01, p23, eps0, eps1)


def _ni_of(degp_blk):
    return lax.rsqrt(jnp.maximum(degp_blk[0][:, :1], 1.0))


def _no_of(degp_blk):
    return lax.rsqrt(jnp.maximum(degp_blk[1][:, :1], 1.0))


_DEG_SPEC = pl.BlockSpec((2, RB, 16), lambda i: (0, i, 0))
_AGG_SPEC = pl.BlockSpec((RB, F), lambda i: (i, 0))
_ROW_SPEC = pl.BlockSpec((RB, F), lambda i: (i, 0))


def _xs_body(x_ref, degp_ref, o_ref):
    o_ref[...] = x_ref[...] * _no_of(degp_ref)


def _xs_call(x, degp):
    return pl.pallas_call(
        _xs_body,
        grid=(N // RB,),
        in_specs=[_ROW_SPEC, _DEG_SPEC],
        out_specs=_ROW_SPEC,
        out_shape=jax.ShapeDtypeStruct((N, F), _F32),
    )(x, degp)


def _node_body(agg_ref, degp_ref, w_ref, b_ref, o_ref):
    t = agg_ref[...] * _ni_of(degp_ref)
    y = jnp.dot(t, w_ref[...], preferred_element_type=_F32) + b_ref[...]
    y = jnp.maximum(y, 0.0) * _no_of(degp_ref)
    o_ref[...] = y


def _node_call(agg, degp, w, b):
    return pl.pallas_call(
        _node_body,
        grid=(N // RB,),
        in_specs=[
            _AGG_SPEC,
            _DEG_SPEC,
            pl.BlockSpec((F, F), lambda i: (0, 0)),
            pl.BlockSpec((1, F), lambda i: (0, 0)),
        ],
        out_specs=_ROW_SPEC,
        out_shape=jax.ShapeDtypeStruct((N, F), _F32),
    )(agg, degp, w, b.reshape(1, F))


def _proj_body(agg_ref, degp_ref, w1_ref, b1_ref, ws_ref, wd_ref, bc_ref,
               p01_ref, p23_ref):
    t = agg_ref[...] * _ni_of(degp_ref)
    z = jnp.dot(t, w1_ref[...], preferred_element_type=_F32) + b1_ref[...]
    z = jnp.maximum(z, 0.0)
    ps = jnp.exp(jnp.dot(z, ws_ref[...], preferred_element_type=_F32))
    pd = jnp.exp(
        jnp.dot(z, wd_ref[...], preferred_element_type=_F32) + bc_ref[...])
    p01_ref[0] = ps[:, :2 * F]
    p01_ref[1] = pd[:, :2 * F]
    p23_ref[0] = ps[:, 2 * F:]
    p23_ref[1] = pd[:, 2 * F:]


def _proj_call(agg, degp, w1, b1, ws, wd, bc):
    spec_p = pl.BlockSpec((2, RB, 2 * F), lambda i: (0, i, 0))
    return pl.pallas_call(
        _proj_body,
        grid=(N // RB,),
        in_specs=[
            _AGG_SPEC,
            _DEG_SPEC,
            pl.BlockSpec((F, F), lambda i: (0, 0)),
            pl.BlockSpec((1, F), lambda i: (0, 0)),
            pl.BlockSpec((F, 4 * F), lambda i: (0, 0)),
            pl.BlockSpec((F, 4 * F), lambda i: (0, 0)),
            pl.BlockSpec((1, 4 * F), lambda i: (0, 0)),
        ],
        out_specs=(spec_p, spec_p),
        out_shape=(jax.ShapeDtypeStruct((2, N, 2 * F), _F32),
                   jax.ShapeDtypeStruct((2, N, 2 * F), _F32)),
    )(agg, degp, w1, b1.reshape(1, F), ws, wd, bc.reshape(1, 4 * F))


def _final_body(agg_ref, degp_ref, w_ref, b_ref, o_ref):
    t = agg_ref[...] * _ni_of(degp_ref)
    y = jnp.dot(t, w_ref[...], preferred_element_type=_F32) + b_ref[...]
    m = jnp.max(y, axis=-1, keepdims=True)
    ey = jnp.exp(y - m)
    o_ref[...] = ey / jnp.sum(ey, axis=-1, keepdims=True)


def _final_call(agg, degp, w, b):
    return pl.pallas_call(
        _final_body,
        grid=(N // RB,),
        in_specs=[
            _AGG_SPEC,
            _DEG_SPEC,
            pl.BlockSpec((F, F), lambda i: (0, 0)),
            pl.BlockSpec((1, F), lambda i: (0, 0)),
        ],
        out_specs=_ROW_SPEC,
        out_shape=jax.ShapeDtypeStruct((N, F), _F32),
    )(agg, degp, w, b.reshape(1, F))


def kernel(x, edge_index, W_enc0, b_enc0, W_enc1, b_enc1,
           W_gn0, b_gn0, W_gn1, b_gn1,
           Wmu0, bmu0, Wls0, bls0, Wmu1, bmu1, Wls1, bls1,
           eps0, eps1):
    src = edge_index[0]
    dst = edge_index[1]
    sidxT = src.reshape(NS, NCH, C)
    didxT = dst.reshape(NS, NCH, C)
    sidxW = src.reshape(NW, NCHM, CM)
    didxW = dst.reshape(NW, NCHM, CM)
    sdidx = jnp.concatenate([sidxW, didxW + N], axis=-1)

    ws_cat = jnp.concatenate(
        [Wmu0[:F], Wls0[:F], Wmu1[:F], Wls1[:F]], axis=1)
    wd_cat = jnp.concatenate(
        [Wmu0[F:], Wls0[F:], Wmu1[F:], Wls1[F:]], axis=1)
    bc = jnp.concatenate([bmu0, bls0, bmu1, bls1])

    degp = _deg_call(sidxT, didxT)
    xs = _xs_call(x, degp)
    agg1 = _conv_call(sidxT, didxT, xs)
    z1s = _node_call(agg1, degp, W_enc0, b_enc0)
    agg2 = _conv_call(sidxT, didxT, z1s)
    p01, p23 = _proj_call(agg2, degp, W_enc1, b_enc1, ws_cat, wd_cat, bc)
    a0, a1 = _mlp_call(sdidx, p01.reshape(2 * N, 2 * F),
                       p23.reshape(2 * N, 2 * F), eps0, eps1)
    agg3 = _conv_call(sidxT, didxT, xs, a=a0)
    h1s = _node_call(agg3, degp, W_gn0, b_gn0)
    agg4 = _conv_call(sidxT, didxT, h1s, a=a1)
    return _final_call(agg4, degp, W_gn1, b_gn1)
